# Initial kernel scaffold; baseline (speedup 1.0000x reference)
import jax
import jax.numpy as jnp
from jax.experimental import pallas as pl


def kernel(h, edge_index, coord, edge_attr, We_w1, We_b1, We_w2, We_b2, Wn_w1, Wn_b1, Wn_w2, Wn_b2, Wc_w1, Wc_b1, Wc_w2):
    raise NotImplementedError("write your pallas kernel here")


# speedup vs baseline: 3.7011x; 3.7011x over previous
import functools

import jax
import jax.numpy as jnp
from jax import lax
from jax.experimental import pallas as pl
from jax.experimental.pallas import tpu as pltpu
from jax.experimental.pallas import tpu_sc as plsc

D = 128
H = 128
DE = 16

NC = 2
NS = 16
L = 16
NW = NC * NS

CH = 256


def _silu(x):
    return x * (1.0 / (1.0 + jnp.exp(-x)))


def _pre_body(h_ref, wa_ref, wb_ref, a_ref, b_ref):
    hb = h_ref[...]
    a_ref[...] = jnp.dot(hb, wa_ref[...], preferred_element_type=jnp.float32)
    b_ref[...] = jnp.dot(hb, wb_ref[...], preferred_element_type=jnp.float32)


def _precompute(h, wa, wb):
    n = h.shape[0]
    bn = 2000
    return pl.pallas_call(
        _pre_body,
        grid=(n // bn,),
        in_specs=[
            pl.BlockSpec((bn, D), lambda i: (i, 0)),
            pl.BlockSpec((D, H), lambda i: (0, 0)),
            pl.BlockSpec((D, H), lambda i: (0, 0)),
        ],
        out_specs=[
            pl.BlockSpec((bn, H), lambda i: (i, 0)),
            pl.BlockSpec((bn, H), lambda i: (i, 0)),
        ],
        out_shape=[
            jax.ShapeDtypeStruct((n, H), jnp.float32),
            jax.ShapeDtypeStruct((n, H), jnp.float32),
        ],
    )(h, wa, wb)


def _sc_gather(A, B, row, col, cx, cy, cz):
    e = row.shape[0]
    n = A.shape[0]
    nchunk = e // CH
    nit = (nchunk + NW - 1) // NW
    mesh = plsc.VectorSubcoreMesh(core_axis_name="c", subcore_axis_name="s")

    @functools.partial(
        pl.kernel,
        out_type=[
            jax.ShapeDtypeStruct((e, H), jnp.float32),
            jax.ShapeDtypeStruct((e,), jnp.float32),
            jax.ShapeDtypeStruct((e,), jnp.float32),
            jax.ShapeDtypeStruct((e,), jnp.float32),
            jax.ShapeDtypeStruct((e,), jnp.float32),
        ],
        mesh=mesh,
        scratch_types=[
            pltpu.VMEM((2, 128), jnp.int32),
            pltpu.VMEM((2, 128), jnp.int32),
            pltpu.VMEM((CH, H), jnp.float32),
            pltpu.VMEM((CH, H), jnp.float32),
            pltpu.VMEM((n,), jnp.float32),
            pltpu.VMEM((n,), jnp.float32),
            pltpu.VMEM((n,), jnp.float32),
            pltpu.VMEM((CH,), jnp.float32),
            pltpu.VMEM((CH,), jnp.float32),
            pltpu.VMEM((CH,), jnp.float32),
            pltpu.VMEM((CH,), jnp.float32),
            pltpu.SemaphoreType.DMA,
        ],
        compiler_params=pltpu.CompilerParams(needs_layout_passes=False),
    )
    def k(a_h, b_h, row_h, col_h, cx_h, cy_h, cz_h,
          p_h, dx_h, dy_h, dz_h, rad_h,
          rbuf, cbuf, abuf, bbuf, cxt, cyt, czt, dxb, dyb, dzb, rdb, sem):
        w = lax.axis_index("s") * NC + lax.axis_index("c")
        pltpu.sync_copy(cx_h, cxt)
        pltpu.sync_copy(cy_h, cyt)
        pltpu.sync_copy(cz_h, czt)

        def chunk(i, _):
            c = lax.rem(w + NW * i, nchunk)
            base = c * CH
            for j in range(2):
                pltpu.sync_copy(row_h.at[pl.ds(base + 128 * j, 128)], rbuf.at[j])
                pltpu.sync_copy(col_h.at[pl.ds(base + 128 * j, 128)], cbuf.at[j])
            ds = []
            for j in range(2):
                ds.append(pltpu.async_copy(
                    a_h.at[rbuf.at[j]], abuf.at[pl.ds(128 * j, 128)], sem))
                ds.append(pltpu.async_copy(
                    b_h.at[cbuf.at[j]], bbuf.at[pl.ds(128 * j, 128)], sem))
            for d in ds:
                d.wait()

            def addrow(r, _):
                for j in range(H // L):
                    sl = pl.ds(j * L, L)
                    abuf[r, sl] = abuf[r, sl] + bbuf[r, sl]
                return 0
            lax.fori_loop(0, CH, addrow, 0)

            for g in range(CH // L):
                j = g // 8
                sl16 = pl.ds((g % 8) * L, L)
                sl = pl.ds(g * L, L)
                ri = rbuf[j, sl16]
                ci = cbuf[j, sl16]
                xv = plsc.load_gather(cxt, [ri]) - plsc.load_gather(cxt, [ci])
                yv = plsc.load_gather(cyt, [ri]) - plsc.load_gather(cyt, [ci])
                zv = plsc.load_gather(czt, [ri]) - plsc.load_gather(czt, [ci])
                dxb[sl] = xv
                dyb[sl] = yv
                dzb[sl] = zv
                rdb[sl] = xv * xv + yv * yv + zv * zv

            pltpu.sync_copy(abuf, p_h.at[pl.ds(base, CH)])
            pltpu.sync_copy(dxb, dx_h.at[pl.ds(base, CH)])
            pltpu.sync_copy(dyb, dy_h.at[pl.ds(base, CH)])
            pltpu.sync_copy(dzb, dz_h.at[pl.ds(base, CH)])
            pltpu.sync_copy(rdb, rad_h.at[pl.ds(base, CH)])
            return 0

        lax.fori_loop(0, nit, chunk, 0)

    return k(A, B, row, col, cx, cy, cz)


def _edge_body(p_ref, rad_ref, ea_ref, we_ref, wr_ref, b1_ref, w2_ref, b2_ref,
               wc1_ref, bc1_ref, wc2_ref, m_ref, t_ref):
    rad = rad_ref[...]
    pre = (p_ref[...]
           + jnp.dot(ea_ref[...], we_ref[...], preferred_element_type=jnp.float32)
           + rad * wr_ref[...] + b1_ref[...])
    m1 = _silu(pre)
    m = _silu(jnp.dot(m1, w2_ref[...], preferred_element_type=jnp.float32)
              + b2_ref[...])
    ch = _silu(jnp.dot(m, wc1_ref[...], preferred_element_type=jnp.float32)
               + bc1_ref[...])
    s = jnp.dot(ch, wc2_ref[...], preferred_element_type=jnp.float32)
    m_ref[...] = m
    t_ref[...] = s * lax.rsqrt(rad + 1e-8)


def _edge_mlp(P, rad2, ea, we, wr, b1, w2, b2, wc1, bc1, wc2):
    e = P.shape[0]
    be = 2000
    full = lambda shape: pl.BlockSpec(shape, lambda i: tuple(0 for _ in shape))
    return pl.pallas_call(
        _edge_body,
        grid=(e // be,),
        in_specs=[
            pl.BlockSpec((be, H), lambda i: (i, 0)),
            pl.BlockSpec((be, 1), lambda i: (i, 0)),
            pl.BlockSpec((be, DE), lambda i: (i, 0)),
            full((DE, H)),
            full((1, H)),
            full((1, H)),
            full((H, H)),
            full((1, H)),
            full((H, H)),
            full((1, H)),
            full((H, 1)),
        ],
        out_specs=[
            pl.BlockSpec((be, H), lambda i: (i, 0)),
            pl.BlockSpec((be, 1), lambda i: (i, 0)),
        ],
        out_shape=[
            jax.ShapeDtypeStruct((e, H), jnp.float32),
            jax.ShapeDtypeStruct((e, 1), jnp.float32),
        ],
    )(P, rad2, ea, we, wr, b1, w2, b2, wc1, bc1, wc2)


def _cdiv(a, b):
    return (a + b - 1) // b


RB = 200
ZB = 80


DXW = 16
CHS = 128


def _sc_scatter(m, row, n):
    e = row.shape[0]
    nchunk = e // CHS
    nzb = n // ZB
    mesh = plsc.VectorSubcoreMesh(core_axis_name="c", subcore_axis_name="s")

    @functools.partial(
        pl.kernel,
        out_type=[
            jax.ShapeDtypeStruct((NC, n, H), jnp.float32),
        ],
        mesh=mesh,
        scratch_types=[
            pltpu.VMEM((CHS, H), jnp.float32),
            pltpu.VMEM((1, 128), jnp.int32),
            pltpu.VMEM_SHARED((n, H), jnp.float32),
        ],
        compiler_params=pltpu.CompilerParams(needs_layout_passes=False),
    )
    def k(m_h, row_h, aggp_h, mbuf, ibuf, agg_acc):
        cid = lax.axis_index("c")
        sid = lax.axis_index("s")
        w = sid * NC + cid

        zv = jnp.zeros((L,), jnp.float32)

        def zrow(r, _):
            for j in range(H // L):
                mbuf[r, pl.ds(j * L, L)] = zv
            return 0
        lax.fori_loop(0, CHS, zrow, 0)

        nz_w = lax.div(nzb - 1 - sid, NS) + 1

        def zchunk(i, _):
            kk = sid + NS * i
            pltpu.sync_copy(mbuf.at[pl.ds(0, ZB)],
                            agg_acc.at[pl.ds(kk * ZB, ZB)])
            return 0
        lax.fori_loop(0, nz_w, zchunk, 0)
        plsc.subcore_barrier()

        nc_w = lax.div(nchunk - 1 - w, NW) + 1

        def chunk(i, _):
            c = w + NW * i
            base = c * CHS
            pltpu.sync_copy(row_h.at[pl.ds(base, CHS)], ibuf.at[0])
            pltpu.sync_copy(m_h.at[pl.ds(base, CHS)], mbuf)
            pltpu.sync_copy(mbuf, agg_acc.at[ibuf.at[0]], add=True)
            return 0

        lax.fori_loop(0, nc_w, chunk, 0)
        plsc.subcore_barrier()

        def wchunk(i, _):
            kk = sid + NS * i
            pltpu.sync_copy(agg_acc.at[pl.ds(kk * ZB, ZB)],
                            mbuf.at[pl.ds(0, ZB)])
            pltpu.sync_copy(mbuf.at[pl.ds(0, ZB)],
                            aggp_h.at[cid, pl.ds(kk * ZB, ZB)])
            return 0
        lax.fori_loop(0, nz_w, wchunk, 0)

    (aggp,) = k(m, row)
    return aggp


CHQ = 128


def _sc_dx(row, t, cdx, cdy, cdz, n):
    e = row.shape[0]
    nchunk = e // CHQ
    mesh = plsc.VectorSubcoreMesh(core_axis_name="c", subcore_axis_name="s")

    @functools.partial(
        pl.kernel,
        out_type=[
            jax.ShapeDtypeStruct((NW * n * 4,), jnp.float32),
        ],
        mesh=mesh,
        scratch_types=[
            pltpu.VMEM((CHQ,), jnp.int32),
            pltpu.VMEM((CHQ,), jnp.float32),
            pltpu.VMEM((CHQ,), jnp.float32),
            pltpu.VMEM((CHQ,), jnp.float32),
            pltpu.VMEM((CHQ,), jnp.float32),
            pltpu.VMEM((n * 4,), jnp.float32),
        ],
        compiler_params=pltpu.CompilerParams(needs_layout_passes=False),
    )
    def k(row_h, t_h, x_h, y_h, z_h, dxp_h, ib, tb, xb, yb, zb, acc):
        cid = lax.axis_index("c")
        sid = lax.axis_index("s")
        w = sid * NC + cid

        zv = jnp.zeros((L,), jnp.float32)

        def zflat(i, _):
            acc[pl.ds(i * L, L)] = zv
            return 0
        lax.fori_loop(0, n * 4 // L, zflat, 0)

        iota = lax.iota(jnp.int32, L)
        qoff = lax.div(iota, 4)
        cmod = lax.rem(iota, 4)

        nc_w = lax.div(nchunk - 1 - w, NW) + 1

        def chunk(i, _):
            base = (w + NW * i) * CHQ
            pltpu.sync_copy(row_h.at[pl.ds(base, CHQ)], ib)
            pltpu.sync_copy(t_h.at[pl.ds(base, CHQ)], tb)
            pltpu.sync_copy(x_h.at[pl.ds(base, CHQ)], xb)
            pltpu.sync_copy(y_h.at[pl.ds(base, CHQ)], yb)
            pltpu.sync_copy(z_h.at[pl.ds(base, CHQ)], zb)
            for q in range(CHQ // 4):
                qidx = qoff + q * 4
                rv = plsc.load_gather(ib, [qidx])
                tv = plsc.load_gather(tb, [qidx])
                xv = plsc.load_gather(xb, [qidx])
                yv = plsc.load_gather(yb, [qidx])
                zv2 = plsc.load_gather(zb, [qidx])
                val = tv * jnp.where(
                    cmod == 0, xv,
                    jnp.where(cmod == 1, yv,
                              jnp.where(cmod == 2, zv2,
                                        jnp.zeros((L,), jnp.float32))))
                plsc.addupdate_scatter(acc, [rv * 4 + cmod], val)
            return 0

        lax.fori_loop(0, nc_w, chunk, 0)
        pltpu.sync_copy(acc, dxp_h.at[pl.ds(w * n * 4, n * 4)])

    (dxp,) = k(row, t, cdx, cdy, cdz)
    return dxp.reshape(NW, n * 4)


def _dxr_body(dxp_ref, out_ref):
    acc = dxp_ref[0]
    for i in range(1, NW):
        acc = acc + dxp_ref[i]
    out_ref[...] = acc


def _dx_reduce(dxp_flat, n):
    total = dxp_flat.shape[1]
    return pl.pallas_call(
        _dxr_body,
        grid=(1,),
        in_specs=[pl.BlockSpec((NW, total), lambda i: (0, 0))],
        out_specs=pl.BlockSpec((total,), lambda i: (0,)),
        out_shape=jax.ShapeDtypeStruct((total,), jnp.float32),
    )(dxp_flat)


def _node_body(h_ref, coord_ref, aggp_ref, dx_ref, wn1_ref, bn1_ref,
               wn2_ref, bn2_ref, ho_ref, co_ref):
    hb = h_ref[...]
    agg = aggp_ref[0] + aggp_ref[1]
    z = _silu(jnp.dot(hb, wn1_ref[0:D, :], preferred_element_type=jnp.float32)
              + jnp.dot(agg, wn1_ref[D:2 * D, :], preferred_element_type=jnp.float32)
              + bn1_ref[...])
    ho_ref[...] = hb + jnp.dot(z, wn2_ref[...], preferred_element_type=jnp.float32) \
        + bn2_ref[...]
    co_ref[...] = coord_ref[...] + dx_ref[:, 0:3]


def _node_mlp(h, coord, aggp, dxp, wn1, bn1, wn2, bn2):
    n = h.shape[0]
    bn = 2000
    full = lambda shape: pl.BlockSpec(shape, lambda i: tuple(0 for _ in shape))
    return pl.pallas_call(
        _node_body,
        grid=(n // bn,),
        in_specs=[
            pl.BlockSpec((bn, D), lambda i: (i, 0)),
            pl.BlockSpec((bn, 3), lambda i: (i, 0)),
            pl.BlockSpec((NC, bn, H), lambda i: (0, i, 0)),
            pl.BlockSpec((bn, 4), lambda i: (i, 0)),
            full((2 * D, H)),
            full((1, H)),
            full((H, D)),
            full((1, D)),
        ],
        out_specs=[
            pl.BlockSpec((bn, D), lambda i: (i, 0)),
            pl.BlockSpec((bn, 3), lambda i: (i, 0)),
        ],
        out_shape=[
            jax.ShapeDtypeStruct((n, D), jnp.float32),
            jax.ShapeDtypeStruct((n, 3), jnp.float32),
        ],
    )(h, coord, aggp, dxp, wn1, bn1, wn2, bn2)


def kernel(h, edge_index, coord, edge_attr,
           We_w1, We_b1, We_w2, We_b2,
           Wn_w1, Wn_b1, Wn_w2, Wn_b2,
           Wc_w1, Wc_b1, Wc_w2):
    n = h.shape[0]
    e = edge_index.shape[1]
    row = edge_index[0]
    col = edge_index[1]

    wa = We_w1[0:D, :]
    wb = We_w1[D:2 * D, :]
    wr = We_w1[2 * D:2 * D + 1, :]
    we = We_w1[2 * D + 1:, :]

    A, B = _precompute(h, wa, wb)
    cx = coord[:, 0]
    cy = coord[:, 1]
    cz = coord[:, 2]

    P, cdx, cdy, cdz, rad = _sc_gather(A, B, row, col, cx, cy, cz)

    m, t = _edge_mlp(P, rad.reshape(e, 1), edge_attr, we, wr,
                     We_b1.reshape(1, H), We_w2, We_b2.reshape(1, H),
                     Wc_w1, Wc_b1.reshape(1, H), Wc_w2)

    aggp = _sc_scatter(m, row, n)
    dxp_flat = _sc_dx(row, t.reshape(e), cdx, cdy, cdz, n)
    dx2 = _dx_reduce(dxp_flat, n).reshape(n, 4)

    h_out, coord_out = _node_mlp(h, coord, aggp, dx2, Wn_w1,
                                 Wn_b1.reshape(1, H), Wn_w2,
                                 Wn_b2.reshape(1, D))
    return h_out, coord_out, m


# speedup vs baseline: 3.8295x; 1.0347x over previous
import functools

import jax
import jax.numpy as jnp
from jax import lax
from jax.experimental import pallas as pl
from jax.experimental.pallas import tpu as pltpu
from jax.experimental.pallas import tpu_sc as plsc

D = 128
H = 128
DE = 16

NC = 2
NS = 16
L = 16
NW = NC * NS

CH = 256


def _silu(x):
    return 0.5 * x * (1.0 + jnp.tanh(0.5 * x))


def _pre_body(h_ref, wa_ref, wb_ref, a_ref, b_ref):
    hb = h_ref[...]
    a_ref[...] = jnp.dot(hb, wa_ref[...], preferred_element_type=jnp.float32)
    b_ref[...] = jnp.dot(hb, wb_ref[...], preferred_element_type=jnp.float32)


def _precompute(h, wa, wb):
    n = h.shape[0]
    bn = 2000
    return pl.pallas_call(
        _pre_body,
        grid=(n // bn,),
        in_specs=[
            pl.BlockSpec((bn, D), lambda i: (i, 0)),
            pl.BlockSpec((D, H), lambda i: (0, 0)),
            pl.BlockSpec((D, H), lambda i: (0, 0)),
        ],
        out_specs=[
            pl.BlockSpec((bn, H), lambda i: (i, 0)),
            pl.BlockSpec((bn, H), lambda i: (i, 0)),
        ],
        out_shape=[
            jax.ShapeDtypeStruct((n, H), jnp.float32),
            jax.ShapeDtypeStruct((n, H), jnp.float32),
        ],
    )(h, wa, wb)


def _sc_gather(A, B, row, col, cx, cy, cz):
    e = row.shape[0]
    n = A.shape[0]
    nchunk = e // CH
    nit = (nchunk + NW - 1) // NW
    mesh = plsc.VectorSubcoreMesh(core_axis_name="c", subcore_axis_name="s")

    @functools.partial(
        pl.kernel,
        out_type=[
            jax.ShapeDtypeStruct((e, H), jnp.float32),
            jax.ShapeDtypeStruct((e,), jnp.float32),
            jax.ShapeDtypeStruct((e,), jnp.float32),
            jax.ShapeDtypeStruct((e,), jnp.float32),
            jax.ShapeDtypeStruct((e,), jnp.float32),
        ],
        mesh=mesh,
        scratch_types=[
            pltpu.VMEM((2, 128), jnp.int32),
            pltpu.VMEM((2, 128), jnp.int32),
            pltpu.VMEM((CH, H), jnp.float32),
            pltpu.VMEM((CH, H), jnp.float32),
            pltpu.VMEM((n,), jnp.float32),
            pltpu.VMEM((n,), jnp.float32),
            pltpu.VMEM((n,), jnp.float32),
            pltpu.VMEM((CH,), jnp.float32),
            pltpu.VMEM((CH,), jnp.float32),
            pltpu.VMEM((CH,), jnp.float32),
            pltpu.VMEM((CH,), jnp.float32),
            pltpu.SemaphoreType.DMA,
        ],
        compiler_params=pltpu.CompilerParams(needs_layout_passes=False),
    )
    def k(a_h, b_h, row_h, col_h, cx_h, cy_h, cz_h,
          p_h, dx_h, dy_h, dz_h, rad_h,
          rbuf, cbuf, abuf, bbuf, cxt, cyt, czt, dxb, dyb, dzb, rdb, sem):
        w = lax.axis_index("s") * NC + lax.axis_index("c")
        pltpu.sync_copy(cx_h, cxt)
        pltpu.sync_copy(cy_h, cyt)
        pltpu.sync_copy(cz_h, czt)

        def chunk(i, _):
            c = lax.rem(w + NW * i, nchunk)
            base = c * CH
            for j in range(2):
                pltpu.sync_copy(row_h.at[pl.ds(base + 128 * j, 128)], rbuf.at[j])
                pltpu.sync_copy(col_h.at[pl.ds(base + 128 * j, 128)], cbuf.at[j])
            ds = []
            for j in range(2):
                ds.append(pltpu.async_copy(
                    a_h.at[rbuf.at[j]], abuf.at[pl.ds(128 * j, 128)], sem))
                ds.append(pltpu.async_copy(
                    b_h.at[cbuf.at[j]], bbuf.at[pl.ds(128 * j, 128)], sem))
            for d in ds:
                d.wait()

            def addrow(r, _):
                for j in range(H // L):
                    sl = pl.ds(j * L, L)
                    abuf[r, sl] = abuf[r, sl] + bbuf[r, sl]
                return 0
            lax.fori_loop(0, CH, addrow, 0)

            for g in range(CH // L):
                j = g // 8
                sl16 = pl.ds((g % 8) * L, L)
                sl = pl.ds(g * L, L)
                ri = rbuf[j, sl16]
                ci = cbuf[j, sl16]
                xv = plsc.load_gather(cxt, [ri]) - plsc.load_gather(cxt, [ci])
                yv = plsc.load_gather(cyt, [ri]) - plsc.load_gather(cyt, [ci])
                zv = plsc.load_gather(czt, [ri]) - plsc.load_gather(czt, [ci])
                dxb[sl] = xv
                dyb[sl] = yv
                dzb[sl] = zv
                rdb[sl] = xv * xv + yv * yv + zv * zv

            pltpu.sync_copy(abuf, p_h.at[pl.ds(base, CH)])
            pltpu.sync_copy(dxb, dx_h.at[pl.ds(base, CH)])
            pltpu.sync_copy(dyb, dy_h.at[pl.ds(base, CH)])
            pltpu.sync_copy(dzb, dz_h.at[pl.ds(base, CH)])
            pltpu.sync_copy(rdb, rad_h.at[pl.ds(base, CH)])
            return 0

        lax.fori_loop(0, nit, chunk, 0)

    return k(A, B, row, col, cx, cy, cz)


def _edge_body(p_ref, rad_ref, ea_ref, cds_ref, we_ref, wr_ref, b1_ref,
               w2_ref, b2_ref, wc1_ref, bc1_ref, wc2_ref, m_ref, tr_ref):
    rad = rad_ref[...]
    pre = (p_ref[...]
           + jnp.dot(ea_ref[...], we_ref[...], preferred_element_type=jnp.float32)
           + rad * wr_ref[...] + b1_ref[...])
    m1 = _silu(pre)
    m = _silu(jnp.dot(m1, w2_ref[...], preferred_element_type=jnp.float32)
              + b2_ref[...])
    ch = _silu(jnp.dot(m, wc1_ref[...], preferred_element_type=jnp.float32)
               + bc1_ref[...])
    s = jnp.dot(ch, wc2_ref[...], preferred_element_type=jnp.float32)
    m_ref[...] = m
    tr_ref[...] = cds_ref[...] * (s * lax.rsqrt(rad + 1e-8))


def _edge_mlp(P, rad2, ea, cds, we, wr, b1, w2, b2, wc1, bc1, wc2):
    e = P.shape[0]
    be = 2000
    full = lambda shape: pl.BlockSpec(shape, lambda i: tuple(0 for _ in shape))
    return pl.pallas_call(
        _edge_body,
        grid=(e // be,),
        in_specs=[
            pl.BlockSpec((be, H), lambda i: (i, 0)),
            pl.BlockSpec((be, 1), lambda i: (i, 0)),
            pl.BlockSpec((be, DE), lambda i: (i, 0)),
            pl.BlockSpec((be, 4), lambda i: (i, 0)),
            full((DE, H)),
            full((1, H)),
            full((1, H)),
            full((H, H)),
            full((1, H)),
            full((H, H)),
            full((1, H)),
            full((H, 1)),
        ],
        out_specs=[
            pl.BlockSpec((be, H), lambda i: (i, 0)),
            pl.BlockSpec((be, 4), lambda i: (i, 0)),
        ],
        out_shape=[
            jax.ShapeDtypeStruct((e, H), jnp.float32),
            jax.ShapeDtypeStruct((e, 4), jnp.float32),
        ],
    )(P, rad2, ea, cds, we, wr, b1, w2, b2, wc1, bc1, wc2)


def _cdiv(a, b):
    return (a + b - 1) // b


RB = 200
ZB = 80


DXW = 16
CHS = 128


def _sc_scatter(m, row, n):
    e = row.shape[0]
    nchunk = e // CHS
    nzb = n // ZB
    mesh = plsc.VectorSubcoreMesh(core_axis_name="c", subcore_axis_name="s")

    @functools.partial(
        pl.kernel,
        out_type=[
            jax.ShapeDtypeStruct((NC, n, H), jnp.float32),
        ],
        mesh=mesh,
        scratch_types=[
            pltpu.VMEM((CHS, H), jnp.float32),
            pltpu.VMEM((1, 128), jnp.int32),
            pltpu.VMEM_SHARED((n, H), jnp.float32),
        ],
        compiler_params=pltpu.CompilerParams(needs_layout_passes=False),
    )
    def k(m_h, row_h, aggp_h, mbuf, ibuf, agg_acc):
        cid = lax.axis_index("c")
        sid = lax.axis_index("s")
        w = sid * NC + cid

        zv = jnp.zeros((L,), jnp.float32)

        def zrow(r, _):
            for j in range(H // L):
                mbuf[r, pl.ds(j * L, L)] = zv
            return 0
        lax.fori_loop(0, CHS, zrow, 0)

        nz_w = lax.div(nzb - 1 - sid, NS) + 1

        def zchunk(i, _):
            kk = sid + NS * i
            pltpu.sync_copy(mbuf.at[pl.ds(0, ZB)],
                            agg_acc.at[pl.ds(kk * ZB, ZB)])
            return 0
        lax.fori_loop(0, nz_w, zchunk, 0)
        plsc.subcore_barrier()

        nc_w = lax.div(nchunk - 1 - w, NW) + 1

        def chunk(i, _):
            c = w + NW * i
            base = c * CHS
            pltpu.sync_copy(row_h.at[pl.ds(base, CHS)], ibuf.at[0])
            pltpu.sync_copy(m_h.at[pl.ds(base, CHS)], mbuf)
            pltpu.sync_copy(mbuf, agg_acc.at[ibuf.at[0]], add=True)
            return 0

        lax.fori_loop(0, nc_w, chunk, 0)
        plsc.subcore_barrier()

        def wchunk(i, _):
            kk = sid + NS * i
            pltpu.sync_copy(agg_acc.at[pl.ds(kk * ZB, ZB)],
                            mbuf.at[pl.ds(0, ZB)])
            pltpu.sync_copy(mbuf.at[pl.ds(0, ZB)],
                            aggp_h.at[cid, pl.ds(kk * ZB, ZB)])
            return 0
        lax.fori_loop(0, nz_w, wchunk, 0)

    (aggp,) = k(m, row)
    return aggp


CHQ = 512


def _sc_dx(row, trans4, n):
    e = row.shape[0]
    nchunk = e // CHQ
    mesh = plsc.VectorSubcoreMesh(core_axis_name="c", subcore_axis_name="s")

    @functools.partial(
        pl.kernel,
        out_type=[
            jax.ShapeDtypeStruct((NW * n * 4,), jnp.float32),
        ],
        mesh=mesh,
        scratch_types=[
            pltpu.VMEM((CHQ,), jnp.int32),
            pltpu.VMEM((CHQ, 4), jnp.float32),
            pltpu.VMEM((n * 4,), jnp.float32),
        ],
        compiler_params=pltpu.CompilerParams(needs_layout_passes=False),
    )
    def k(row_h, tr_h, dxp_h, ib, tbuf, acc):
        cid = lax.axis_index("c")
        sid = lax.axis_index("s")
        w = sid * NC + cid

        zv = jnp.zeros((L,), jnp.float32)

        def zflat(i, _):
            acc[pl.ds(i * L, L)] = zv
            return 0
        lax.fori_loop(0, n * 4 // L, zflat, 0)

        iota = lax.iota(jnp.int32, L)
        qoff = lax.div(iota, 4)
        cmod = lax.rem(iota, 4)

        nc_w = lax.div(nchunk - 1 - w, NW) + 1

        def chunk(i, _):
            base = (w + NW * i) * CHQ
            pltpu.sync_copy(row_h.at[pl.ds(base, CHQ)], ib)
            pltpu.sync_copy(tr_h.at[pl.ds(base, CHQ)], tbuf)

            def quad(q, _):
                qidx = qoff + q * 4
                rv = plsc.load_gather(ib, [qidx])
                val = plsc.load_gather(tbuf, [qidx, cmod])
                plsc.addupdate_scatter(acc, [rv * 4 + cmod], val)
                return 0
            lax.fori_loop(0, CHQ // 4, quad, 0)
            return 0

        lax.fori_loop(0, nc_w, chunk, 0)
        pltpu.sync_copy(acc, dxp_h.at[pl.ds(w * n * 4, n * 4)])

    (dxp,) = k(row, trans4)
    return dxp.reshape(NW, n * 4)


def _dxr_body(dxp_ref, out_ref):
    acc = dxp_ref[0]
    for i in range(1, NW):
        acc = acc + dxp_ref[i]
    out_ref[...] = acc


def _dx_reduce(dxp_flat, n):
    total = dxp_flat.shape[1]
    return pl.pallas_call(
        _dxr_body,
        grid=(1,),
        in_specs=[pl.BlockSpec((NW, total), lambda i: (0, 0))],
        out_specs=pl.BlockSpec((total,), lambda i: (0,)),
        out_shape=jax.ShapeDtypeStruct((total,), jnp.float32),
    )(dxp_flat)


def _node_body(h_ref, coord_ref, aggp_ref, dx_ref, wn1_ref, bn1_ref,
               wn2_ref, bn2_ref, ho_ref, co_ref):
    hb = h_ref[...]
    agg = aggp_ref[0] + aggp_ref[1]
    z = _silu(jnp.dot(hb, wn1_ref[0:D, :], preferred_element_type=jnp.float32)
              + jnp.dot(agg, wn1_ref[D:2 * D, :], preferred_element_type=jnp.float32)
              + bn1_ref[...])
    ho_ref[...] = hb + jnp.dot(z, wn2_ref[...], preferred_element_type=jnp.float32) \
        + bn2_ref[...]
    co_ref[...] = coord_ref[...] + dx_ref[:, 0:3]


def _node_mlp(h, coord, aggp, dxp, wn1, bn1, wn2, bn2):
    n = h.shape[0]
    bn = 2000
    full = lambda shape: pl.BlockSpec(shape, lambda i: tuple(0 for _ in shape))
    return pl.pallas_call(
        _node_body,
        grid=(n // bn,),
        in_specs=[
            pl.BlockSpec((bn, D), lambda i: (i, 0)),
            pl.BlockSpec((bn, 3), lambda i: (i, 0)),
            pl.BlockSpec((NC, bn, H), lambda i: (0, i, 0)),
            pl.BlockSpec((bn, 4), lambda i: (i, 0)),
            full((2 * D, H)),
            full((1, H)),
            full((H, D)),
            full((1, D)),
        ],
        out_specs=[
            pl.BlockSpec((bn, D), lambda i: (i, 0)),
            pl.BlockSpec((bn, 3), lambda i: (i, 0)),
        ],
        out_shape=[
            jax.ShapeDtypeStruct((n, D), jnp.float32),
            jax.ShapeDtypeStruct((n, 3), jnp.float32),
        ],
    )(h, coord, aggp, dxp, wn1, bn1, wn2, bn2)


def kernel(h, edge_index, coord, edge_attr,
           We_w1, We_b1, We_w2, We_b2,
           Wn_w1, Wn_b1, Wn_w2, Wn_b2,
           Wc_w1, Wc_b1, Wc_w2):
    n = h.shape[0]
    e = edge_index.shape[1]
    row = edge_index[0]
    col = edge_index[1]

    wa = We_w1[0:D, :]
    wb = We_w1[D:2 * D, :]
    wr = We_w1[2 * D:2 * D + 1, :]
    we = We_w1[2 * D + 1:, :]

    A, B = _precompute(h, wa, wb)
    cx = coord[:, 0]
    cy = coord[:, 1]
    cz = coord[:, 2]

    P, cdx, cdy, cdz, rad = _sc_gather(A, B, row, col, cx, cy, cz)

    cds = jnp.stack([cdx, cdy, cdz, jnp.zeros_like(cdx)], axis=1)
    m, trans4 = _edge_mlp(P, rad.reshape(e, 1), edge_attr, cds, we, wr,
                          We_b1.reshape(1, H), We_w2, We_b2.reshape(1, H),
                          Wc_w1, Wc_b1.reshape(1, H), Wc_w2)

    aggp = _sc_scatter(m, row, n)
    dxp_flat = _sc_dx(row, trans4, n)
    dx2 = _dx_reduce(dxp_flat, n).reshape(n, 4)

    h_out, coord_out = _node_mlp(h, coord, aggp, dx2, Wn_w1,
                                 Wn_b1.reshape(1, H), Wn_w2,
                                 Wn_b2.reshape(1, D))
    return h_out, coord_out, m


# speedup vs baseline: 4.3383x; 1.1329x over previous
import functools

import jax
import jax.numpy as jnp
from jax import lax
from jax.experimental import pallas as pl
from jax.experimental.pallas import tpu as pltpu
from jax.experimental.pallas import tpu_sc as plsc

D = 128
H = 128
DE = 16

NC = 2
NS = 16
L = 16
NW = NC * NS

CH = 256


def _silu(x):
    return 0.5 * x * (1.0 + jnp.tanh(0.5 * x))


def _pre_body(h_ref, wa_ref, wb_ref, a_ref, b_ref):
    hb = h_ref[...]
    a_ref[...] = jnp.dot(hb, wa_ref[...], preferred_element_type=jnp.float32)
    b_ref[...] = jnp.dot(hb, wb_ref[...], preferred_element_type=jnp.float32)


def _precompute(h, wa, wb):
    n = h.shape[0]
    bn = 2000
    return pl.pallas_call(
        _pre_body,
        grid=(n // bn,),
        in_specs=[
            pl.BlockSpec((bn, D), lambda i: (i, 0)),
            pl.BlockSpec((D, H), lambda i: (0, 0)),
            pl.BlockSpec((D, H), lambda i: (0, 0)),
        ],
        out_specs=[
            pl.BlockSpec((bn, H), lambda i: (i, 0)),
            pl.BlockSpec((bn, H), lambda i: (i, 0)),
        ],
        out_shape=[
            jax.ShapeDtypeStruct((n, H), jnp.float32),
            jax.ShapeDtypeStruct((n, H), jnp.float32),
        ],
    )(h, wa, wb)


def _sc_gather(A, B, ei, cx, cy, cz):
    e = ei.shape[1]
    n = A.shape[0]
    nchunk = e // CH
    nit = (nchunk + NW - 1) // NW
    mesh = plsc.VectorSubcoreMesh(core_axis_name="c", subcore_axis_name="s")

    @functools.partial(
        pl.kernel,
        out_type=[
            jax.ShapeDtypeStruct((e, H), jnp.float32),
            jax.ShapeDtypeStruct((4, e), jnp.float32),
        ],
        mesh=mesh,
        scratch_types=[
            pltpu.VMEM((2, CH), jnp.int32),
            pltpu.VMEM((CH, H), jnp.float32),
            pltpu.VMEM((CH, H), jnp.float32),
            pltpu.VMEM((n,), jnp.float32),
            pltpu.VMEM((n,), jnp.float32),
            pltpu.VMEM((n,), jnp.float32),
            pltpu.VMEM((4, CH), jnp.float32),
            pltpu.SemaphoreType.DMA,
        ],
        compiler_params=pltpu.CompilerParams(needs_layout_passes=False),
    )
    def k(a_h, b_h, ei_h, cx_h, cy_h, cz_h,
          p_h, cd_h,
          ibuf, abuf, bbuf, cxt, cyt, czt, buf4, sem):
        w = lax.axis_index("s") * NC + lax.axis_index("c")
        pltpu.sync_copy(cx_h, cxt)
        pltpu.sync_copy(cy_h, cyt)
        pltpu.sync_copy(cz_h, czt)

        def chunk(i, _):
            c = lax.rem(w + NW * i, nchunk)
            base = c * CH
            pltpu.sync_copy(ei_h.at[:, pl.ds(base, CH)], ibuf)
            ds_list = []
            for j in range(CH // 128):
                ds_list.append(pltpu.async_copy(
                    a_h.at[ibuf.at[0, pl.ds(128 * j, 128)]],
                    abuf.at[pl.ds(128 * j, 128)], sem))
                ds_list.append(pltpu.async_copy(
                    b_h.at[ibuf.at[1, pl.ds(128 * j, 128)]],
                    bbuf.at[pl.ds(128 * j, 128)], sem))
            for d in ds_list:
                d.wait()

            def addrow(r, _):
                for j in range(H // L):
                    sl = pl.ds(j * L, L)
                    abuf[r, sl] = abuf[r, sl] + bbuf[r, sl]
                return 0
            lax.fori_loop(0, CH, addrow, 0)

            for g in range(CH // L):
                sl = pl.ds(g * L, L)
                ri = ibuf[0, sl]
                ci = ibuf[1, sl]
                xv = plsc.load_gather(cxt, [ri]) - plsc.load_gather(cxt, [ci])
                yv = plsc.load_gather(cyt, [ri]) - plsc.load_gather(cyt, [ci])
                zv = plsc.load_gather(czt, [ri]) - plsc.load_gather(czt, [ci])
                buf4[0, sl] = xv
                buf4[1, sl] = yv
                buf4[2, sl] = zv
                buf4[3, sl] = xv * xv + yv * yv + zv * zv

            pltpu.sync_copy(abuf, p_h.at[pl.ds(base, CH)])
            pltpu.sync_copy(buf4, cd_h.at[:, pl.ds(base, CH)])
            return 0

        lax.fori_loop(0, nit, chunk, 0)

    return k(A, B, ei, cx, cy, cz)


def _edge_body(p_ref, rad_ref, ea_ref, cds_ref, we_ref, wr_ref, b1_ref,
               w2_ref, b2_ref, wc1_ref, bc1_ref, wc2_ref, m_ref, tr_ref):
    rad = rad_ref[...]
    pre = (p_ref[...].astype(jnp.float32)
           + jnp.dot(ea_ref[...], we_ref[...], preferred_element_type=jnp.float32)
           + rad * wr_ref[...] + b1_ref[...])
    m1 = _silu(pre)
    m = _silu(jnp.dot(m1, w2_ref[...], preferred_element_type=jnp.float32)
              + b2_ref[...])
    ch = _silu(jnp.dot(m, wc1_ref[...], preferred_element_type=jnp.float32)
               + bc1_ref[...])
    s = jnp.dot(ch, wc2_ref[...], preferred_element_type=jnp.float32)
    m_ref[...] = m
    tr_ref[...] = cds_ref[...] * (s * lax.rsqrt(rad + 1e-8))


def _edge_mlp(P, rad2, ea, cds, we, wr, b1, w2, b2, wc1, bc1, wc2):
    e = P.shape[0]
    be = 2000
    full = lambda shape: pl.BlockSpec(shape, lambda i: tuple(0 for _ in shape))
    return pl.pallas_call(
        _edge_body,
        grid=(e // be,),
        in_specs=[
            pl.BlockSpec((be, H), lambda i: (i, 0)),
            pl.BlockSpec((be, 1), lambda i: (i, 0)),
            pl.BlockSpec((be, DE), lambda i: (i, 0)),
            pl.BlockSpec((be, 4), lambda i: (i, 0)),
            full((DE, H)),
            full((1, H)),
            full((1, H)),
            full((H, H)),
            full((1, H)),
            full((H, H)),
            full((1, H)),
            full((H, 1)),
        ],
        out_specs=[
            pl.BlockSpec((be, H), lambda i: (i, 0)),
            pl.BlockSpec((be, 4), lambda i: (i, 0)),
        ],
        out_shape=[
            jax.ShapeDtypeStruct((e, H), jnp.float32),
            jax.ShapeDtypeStruct((e, 4), jnp.float32),
        ],
    )(P, rad2, ea, cds, we, wr, b1, w2, b2, wc1, bc1, wc2)


def _cdiv(a, b):
    return (a + b - 1) // b


RB = 200
ZB = 80


DXW = 16
CHS = 256


def _sc_scatter(m, row, n):
    e = row.shape[0]
    nchunk = e // CHS
    nzb = n // ZB
    mesh = plsc.VectorSubcoreMesh(core_axis_name="c", subcore_axis_name="s")

    @functools.partial(
        pl.kernel,
        out_type=[
            jax.ShapeDtypeStruct((NC, n, H), jnp.float32),
        ],
        mesh=mesh,
        scratch_types=[
            pltpu.VMEM((CHS, H), jnp.float32),
            pltpu.VMEM((2, 128), jnp.int32),
            pltpu.VMEM_SHARED((n, H), jnp.float32),
        ],
        compiler_params=pltpu.CompilerParams(needs_layout_passes=False),
    )
    def k(m_h, row_h, aggp_h, mbuf, ibuf, agg_acc):
        cid = lax.axis_index("c")
        sid = lax.axis_index("s")
        w = sid * NC + cid

        zv = jnp.zeros((L,), jnp.float32)

        def zrow(r, _):
            for j in range(H // L):
                mbuf[r, pl.ds(j * L, L)] = zv
            return 0
        lax.fori_loop(0, CHS, zrow, 0)

        nz_w = lax.div(nzb - 1 - sid, NS) + 1

        def zchunk(i, _):
            kk = sid + NS * i
            pltpu.sync_copy(mbuf.at[pl.ds(0, ZB)],
                            agg_acc.at[pl.ds(kk * ZB, ZB)])
            return 0
        lax.fori_loop(0, nz_w, zchunk, 0)
        plsc.subcore_barrier()

        nc_w = lax.div(nchunk - 1 - w, NW) + 1

        def chunk(i, _):
            c = w + NW * i
            base = c * CHS
            for j in range(2):
                pltpu.sync_copy(row_h.at[pl.ds(base + 128 * j, 128)],
                                ibuf.at[j])
            pltpu.sync_copy(m_h.at[pl.ds(base, CHS)], mbuf)
            for j in range(2):
                pltpu.sync_copy(mbuf.at[pl.ds(128 * j, 128)],
                                agg_acc.at[ibuf.at[j]], add=True)
            return 0

        lax.fori_loop(0, nc_w, chunk, 0)
        plsc.subcore_barrier()

        def wchunk(i, _):
            kk = sid + NS * i
            pltpu.sync_copy(agg_acc.at[pl.ds(kk * ZB, ZB)],
                            mbuf.at[pl.ds(0, ZB)])
            pltpu.sync_copy(mbuf.at[pl.ds(0, ZB)],
                            aggp_h.at[cid, pl.ds(kk * ZB, ZB)])
            return 0
        lax.fori_loop(0, nz_w, wchunk, 0)

    (aggp,) = k(m, row)
    return aggp


CHQ = 512


def _sc_dx(row, trans4, n):
    e = row.shape[0]
    nchunk = e // CHQ
    mesh = plsc.VectorSubcoreMesh(core_axis_name="c", subcore_axis_name="s")

    @functools.partial(
        pl.kernel,
        out_type=[
            jax.ShapeDtypeStruct((NW * n * 4,), jnp.float32),
        ],
        mesh=mesh,
        scratch_types=[
            pltpu.VMEM((CHQ,), jnp.int32),
            pltpu.VMEM((CHQ, 4), jnp.float32),
            pltpu.VMEM((n * 4,), jnp.float32),
        ],
        compiler_params=pltpu.CompilerParams(needs_layout_passes=False),
    )
    def k(row_h, tr_h, dxp_h, ib, tbuf, acc):
        cid = lax.axis_index("c")
        sid = lax.axis_index("s")
        w = sid * NC + cid

        zv = jnp.zeros((L,), jnp.float32)

        def zflat(i, _):
            acc[pl.ds(i * L, L)] = zv
            return 0
        lax.fori_loop(0, n * 4 // L, zflat, 0)

        iota = lax.iota(jnp.int32, L)
        qoff = lax.div(iota, 4)
        cmod = lax.rem(iota, 4)

        nc_w = lax.div(nchunk - 1 - w, NW) + 1

        def chunk(i, _):
            base = (w + NW * i) * CHQ
            pltpu.sync_copy(row_h.at[pl.ds(base, CHQ)], ib)
            pltpu.sync_copy(tr_h.at[pl.ds(base, CHQ)], tbuf)

            def quad(q, _):
                qidx = qoff + q * 4
                rv = plsc.load_gather(ib, [qidx])
                val = plsc.load_gather(tbuf, [qidx, cmod])
                plsc.addupdate_scatter(acc, [rv * 4 + cmod], val)
                return 0
            lax.fori_loop(0, CHQ // 4, quad, 0)
            return 0

        lax.fori_loop(0, nc_w, chunk, 0)
        pltpu.sync_copy(acc, dxp_h.at[pl.ds(w * n * 4, n * 4)])

    (dxp,) = k(row, trans4)
    return dxp.reshape(NW, n * 4)


def _dxr_body(dxp_ref, out_ref):
    acc = dxp_ref[0]
    for i in range(1, NW):
        acc = acc + dxp_ref[i]
    out_ref[...] = acc


def _dx_reduce(dxp_flat, n):
    total = dxp_flat.shape[1]
    return pl.pallas_call(
        _dxr_body,
        grid=(1,),
        in_specs=[pl.BlockSpec((NW, total), lambda i: (0, 0))],
        out_specs=pl.BlockSpec((total,), lambda i: (0,)),
        out_shape=jax.ShapeDtypeStruct((total,), jnp.float32),
    )(dxp_flat)


def _node_body(h_ref, coord_ref, aggp_ref, dx_ref, wn1_ref, bn1_ref,
               wn2_ref, bn2_ref, ho_ref, co_ref):
    hb = h_ref[...]
    agg = aggp_ref[0] + aggp_ref[1]
    z = _silu(jnp.dot(hb, wn1_ref[0:D, :], preferred_element_type=jnp.float32)
              + jnp.dot(agg, wn1_ref[D:2 * D, :], preferred_element_type=jnp.float32)
              + bn1_ref[...])
    ho_ref[...] = hb + jnp.dot(z, wn2_ref[...], preferred_element_type=jnp.float32) \
        + bn2_ref[...]
    co_ref[...] = coord_ref[...] + dx_ref[:, 0:3]


def _node_mlp(h, coord, aggp, dxp, wn1, bn1, wn2, bn2):
    n = h.shape[0]
    bn = 2000
    full = lambda shape: pl.BlockSpec(shape, lambda i: tuple(0 for _ in shape))
    return pl.pallas_call(
        _node_body,
        grid=(n // bn,),
        in_specs=[
            pl.BlockSpec((bn, D), lambda i: (i, 0)),
            pl.BlockSpec((bn, 3), lambda i: (i, 0)),
            pl.BlockSpec((NC, bn, H), lambda i: (0, i, 0)),
            pl.BlockSpec((bn, 4), lambda i: (i, 0)),
            full((2 * D, H)),
            full((1, H)),
            full((H, D)),
            full((1, D)),
        ],
        out_specs=[
            pl.BlockSpec((bn, D), lambda i: (i, 0)),
            pl.BlockSpec((bn, 3), lambda i: (i, 0)),
        ],
        out_shape=[
            jax.ShapeDtypeStruct((n, D), jnp.float32),
            jax.ShapeDtypeStruct((n, 3), jnp.float32),
        ],
    )(h, coord, aggp, dxp, wn1, bn1, wn2, bn2)


def kernel(h, edge_index, coord, edge_attr,
           We_w1, We_b1, We_w2, We_b2,
           Wn_w1, Wn_b1, Wn_w2, Wn_b2,
           Wc_w1, Wc_b1, Wc_w2):
    n = h.shape[0]
    e = edge_index.shape[1]
    row = edge_index[0]
    col = edge_index[1]

    wa = We_w1[0:D, :]
    wb = We_w1[D:2 * D, :]
    wr = We_w1[2 * D:2 * D + 1, :]
    we = We_w1[2 * D + 1:, :]

    A, B = _precompute(h, wa, wb)
    cx = coord[:, 0]
    cy = coord[:, 1]
    cz = coord[:, 2]

    P, cd4 = _sc_gather(A, B, edge_index, cx, cy, cz)
    rad = cd4[3]
    cds = jnp.concatenate(
        [cd4[0:3], jnp.zeros((1, e), jnp.float32)], axis=0).T

    m, trans4 = _edge_mlp(P, rad.reshape(e, 1), edge_attr, cds, we, wr,
                          We_b1.reshape(1, H), We_w2, We_b2.reshape(1, H),
                          Wc_w1, Wc_b1.reshape(1, H), Wc_w2)

    aggp = _sc_scatter(m, row, n)
    dxp_flat = _sc_dx(row, trans4, n)
    dx2 = _dx_reduce(dxp_flat, n).reshape(n, 4)

    h_out, coord_out = _node_mlp(h, coord, aggp, dx2, Wn_w1,
                                 Wn_b1.reshape(1, H), Wn_w2,
                                 Wn_b2.reshape(1, D))
    return h_out, coord_out, m


# speedup vs baseline: 4.6305x; 1.0673x over previous
import functools

import jax
import jax.numpy as jnp
from jax import lax
from jax.experimental import pallas as pl
from jax.experimental.pallas import tpu as pltpu
from jax.experimental.pallas import tpu_sc as plsc

D = 128
H = 128
DE = 16

NC = 2
NS = 16
L = 16
NW = NC * NS

CH = 256


def _silu(x):
    return 0.5 * x * (1.0 + jnp.tanh(0.5 * x))


def _pre_body(h_ref, wa_ref, wb_ref, a_ref, b_ref):
    hb = h_ref[...]
    a_ref[...] = jnp.dot(hb, wa_ref[...], preferred_element_type=jnp.float32)
    b_ref[...] = jnp.dot(hb, wb_ref[...], preferred_element_type=jnp.float32)


def _precompute(h, wa, wb):
    n = h.shape[0]
    bn = 2000
    return pl.pallas_call(
        _pre_body,
        grid=(n // bn,),
        in_specs=[
            pl.BlockSpec((bn, D), lambda i: (i, 0)),
            pl.BlockSpec((D, H), lambda i: (0, 0)),
            pl.BlockSpec((D, H), lambda i: (0, 0)),
        ],
        out_specs=[
            pl.BlockSpec((bn, H), lambda i: (i, 0)),
            pl.BlockSpec((bn, H), lambda i: (i, 0)),
        ],
        out_shape=[
            jax.ShapeDtypeStruct((n, H), jnp.float32),
            jax.ShapeDtypeStruct((n, H), jnp.float32),
        ],
    )(h, wa, wb)


CHG = 128


def _sc_gather(A, B, ei, cx, cy, cz):
    e = ei.shape[1]
    n = A.shape[0]
    nchunk = e // CHG
    nit = (nchunk + NW - 1) // NW
    npair = (nit + 1) // 2
    mesh = plsc.VectorSubcoreMesh(core_axis_name="c", subcore_axis_name="s")

    @functools.partial(
        pl.kernel,
        out_type=[
            jax.ShapeDtypeStruct((e, H), jnp.float32),
            jax.ShapeDtypeStruct((4, e), jnp.float32),
        ],
        mesh=mesh,
        scratch_types=[
            pltpu.VMEM((2, CHG), jnp.int32),
            pltpu.VMEM((2, CHG), jnp.int32),
            pltpu.VMEM((CHG, H), jnp.float32),
            pltpu.VMEM((CHG, H), jnp.float32),
            pltpu.VMEM((CHG, H), jnp.float32),
            pltpu.VMEM((CHG, H), jnp.float32),
            pltpu.VMEM((n,), jnp.float32),
            pltpu.VMEM((n,), jnp.float32),
            pltpu.VMEM((n,), jnp.float32),
            pltpu.VMEM((4, CHG), jnp.float32),
            pltpu.SemaphoreType.DMA,
            pltpu.SemaphoreType.DMA,
        ],
        compiler_params=pltpu.CompilerParams(needs_layout_passes=False),
    )
    def k(a_h, b_h, ei_h, cx_h, cy_h, cz_h,
          p_h, cd_h,
          ibuf0, ibuf1, abuf0, abuf1, bbuf0, bbuf1,
          cxt, cyt, czt, buf4, sem0, sem1):
        w = lax.axis_index("s") * NC + lax.axis_index("c")
        pltpu.sync_copy(cx_h, cxt)
        pltpu.sync_copy(cy_h, cyt)
        pltpu.sync_copy(cz_h, czt)

        sets = ((ibuf0, abuf0, bbuf0, sem0), (ibuf1, abuf1, bbuf1, sem1))

        def fire(c, si):
            ibuf, abuf, bbuf, sem = sets[si]
            base = c * CHG
            pltpu.sync_copy(ei_h.at[:, pl.ds(base, CHG)], ibuf)
            pltpu.async_copy(a_h.at[ibuf.at[0]], abuf, sem)
            pltpu.async_copy(b_h.at[ibuf.at[1]], bbuf, sem)

        def consume(c, si):
            ibuf, abuf, bbuf, sem = sets[si]
            base = c * CHG
            pltpu.make_async_copy(a_h.at[ibuf.at[0]], abuf, sem).wait()
            pltpu.make_async_copy(b_h.at[ibuf.at[1]], bbuf, sem).wait()

            def addrow(r, _):
                for j in range(H // L):
                    sl = pl.ds(j * L, L)
                    abuf[r, sl] = abuf[r, sl] + bbuf[r, sl]
                return 0
            lax.fori_loop(0, CHG, addrow, 0)

            for g in range(CHG // L):
                sl = pl.ds(g * L, L)
                ri = ibuf[0, sl]
                ci = ibuf[1, sl]
                xv = plsc.load_gather(cxt, [ri]) - plsc.load_gather(cxt, [ci])
                yv = plsc.load_gather(cyt, [ri]) - plsc.load_gather(cyt, [ci])
                zv = plsc.load_gather(czt, [ri]) - plsc.load_gather(czt, [ci])
                buf4[0, sl] = xv
                buf4[1, sl] = yv
                buf4[2, sl] = zv
                buf4[3, sl] = xv * xv + yv * yv + zv * zv

            pltpu.sync_copy(abuf, p_h.at[pl.ds(base, CHG)])
            pltpu.sync_copy(buf4, cd_h.at[:, pl.ds(base, CHG)])

        def cidx(i):
            return lax.rem(w + NW * i, nchunk)

        fire(cidx(0), 0)

        def body(p, _):
            fire(cidx(2 * p + 1), 1)
            consume(cidx(2 * p), 0)
            fire(cidx(2 * p + 2), 0)
            consume(cidx(2 * p + 1), 1)
            return 0

        lax.fori_loop(0, npair, body, 0)
        ibuf, abuf, bbuf, sem = sets[0]
        pltpu.make_async_copy(a_h.at[ibuf.at[0]], abuf, sem).wait()
        pltpu.make_async_copy(b_h.at[ibuf.at[1]], bbuf, sem).wait()

    return k(A, B, ei, cx, cy, cz)


def _edge_body(p_ref, rad_ref, ea_ref, cds_ref, we_ref, wr_ref, b1_ref,
               w2_ref, b2_ref, wc1_ref, bc1_ref, wc2_ref, m_ref, tr_ref):
    rad = rad_ref[...]
    pre = (p_ref[...].astype(jnp.float32)
           + jnp.dot(ea_ref[...], we_ref[...], preferred_element_type=jnp.float32)
           + rad * wr_ref[...] + b1_ref[...])
    m1 = _silu(pre)
    m = _silu(jnp.dot(m1, w2_ref[...], preferred_element_type=jnp.float32)
              + b2_ref[...])
    ch = _silu(jnp.dot(m, wc1_ref[...], preferred_element_type=jnp.float32)
               + bc1_ref[...])
    s = jnp.dot(ch, wc2_ref[...], preferred_element_type=jnp.float32)
    m_ref[...] = m
    tr_ref[...] = cds_ref[...] * (s * lax.rsqrt(rad + 1e-8))


def _edge_mlp(P, rad2, ea, cds, we, wr, b1, w2, b2, wc1, bc1, wc2):
    e = P.shape[0]
    be = 2000
    full = lambda shape: pl.BlockSpec(shape, lambda i: tuple(0 for _ in shape))
    return pl.pallas_call(
        _edge_body,
        grid=(e // be,),
        in_specs=[
            pl.BlockSpec((be, H), lambda i: (i, 0)),
            pl.BlockSpec((be, 1), lambda i: (i, 0)),
            pl.BlockSpec((be, DE), lambda i: (i, 0)),
            pl.BlockSpec((be, 4), lambda i: (i, 0)),
            full((DE, H)),
            full((1, H)),
            full((1, H)),
            full((H, H)),
            full((1, H)),
            full((H, H)),
            full((1, H)),
            full((H, 1)),
        ],
        out_specs=[
            pl.BlockSpec((be, H), lambda i: (i, 0)),
            pl.BlockSpec((be, 4), lambda i: (i, 0)),
        ],
        out_shape=[
            jax.ShapeDtypeStruct((e, H), jnp.float32),
            jax.ShapeDtypeStruct((e, 4), jnp.float32),
        ],
    )(P, rad2, ea, cds, we, wr, b1, w2, b2, wc1, bc1, wc2)


def _cdiv(a, b):
    return (a + b - 1) // b


RB = 200
ZB = 80


DXW = 16
CHS = 256


def _sc_scatter(m, row, n):
    e = row.shape[0]
    nchunk = e // CHS
    nzb = n // ZB
    mesh = plsc.VectorSubcoreMesh(core_axis_name="c", subcore_axis_name="s")

    @functools.partial(
        pl.kernel,
        out_type=[
            jax.ShapeDtypeStruct((NC, n, H), jnp.float32),
        ],
        mesh=mesh,
        scratch_types=[
            pltpu.VMEM((CHS, H), jnp.float32),
            pltpu.VMEM((2, 128), jnp.int32),
            pltpu.VMEM_SHARED((n, H), jnp.float32),
        ],
        compiler_params=pltpu.CompilerParams(needs_layout_passes=False),
    )
    def k(m_h, row_h, aggp_h, mbuf, ibuf, agg_acc):
        cid = lax.axis_index("c")
        sid = lax.axis_index("s")
        w = sid * NC + cid

        zv = jnp.zeros((L,), jnp.float32)

        def zrow(r, _):
            for j in range(H // L):
                mbuf[r, pl.ds(j * L, L)] = zv
            return 0
        lax.fori_loop(0, CHS, zrow, 0)

        nz_w = lax.div(nzb - 1 - sid, NS) + 1

        def zchunk(i, _):
            kk = sid + NS * i
            pltpu.sync_copy(mbuf.at[pl.ds(0, ZB)],
                            agg_acc.at[pl.ds(kk * ZB, ZB)])
            return 0
        lax.fori_loop(0, nz_w, zchunk, 0)
        plsc.subcore_barrier()

        nc_w = lax.div(nchunk - 1 - w, NW) + 1

        def chunk(i, _):
            c = w + NW * i
            base = c * CHS
            for j in range(2):
                pltpu.sync_copy(row_h.at[pl.ds(base + 128 * j, 128)],
                                ibuf.at[j])
            pltpu.sync_copy(m_h.at[pl.ds(base, CHS)], mbuf)
            for j in range(2):
                pltpu.sync_copy(mbuf.at[pl.ds(128 * j, 128)],
                                agg_acc.at[ibuf.at[j]], add=True)
            return 0

        lax.fori_loop(0, nc_w, chunk, 0)
        plsc.subcore_barrier()

        def wchunk(i, _):
            kk = sid + NS * i
            pltpu.sync_copy(agg_acc.at[pl.ds(kk * ZB, ZB)],
                            mbuf.at[pl.ds(0, ZB)])
            pltpu.sync_copy(mbuf.at[pl.ds(0, ZB)],
                            aggp_h.at[cid, pl.ds(kk * ZB, ZB)])
            return 0
        lax.fori_loop(0, nz_w, wchunk, 0)

    (aggp,) = k(m, row)
    return aggp


CHQ = 512


def _sc_dx(row, trans4, n):
    e = row.shape[0]
    nchunk = e // CHQ
    mesh = plsc.VectorSubcoreMesh(core_axis_name="c", subcore_axis_name="s")

    @functools.partial(
        pl.kernel,
        out_type=[
            jax.ShapeDtypeStruct((NW * n * 4,), jnp.float32),
        ],
        mesh=mesh,
        scratch_types=[
            pltpu.VMEM((CHQ,), jnp.int32),
            pltpu.VMEM((CHQ, 4), jnp.float32),
            pltpu.VMEM((n * 4,), jnp.float32),
        ],
        compiler_params=pltpu.CompilerParams(needs_layout_passes=False),
    )
    def k(row_h, tr_h, dxp_h, ib, tbuf, acc):
        cid = lax.axis_index("c")
        sid = lax.axis_index("s")
        w = sid * NC + cid

        zv = jnp.zeros((L,), jnp.float32)

        def zflat(i, _):
            acc[pl.ds(i * L, L)] = zv
            return 0
        lax.fori_loop(0, n * 4 // L, zflat, 0)

        iota = lax.iota(jnp.int32, L)
        qoff = lax.div(iota, 4)
        cmod = lax.rem(iota, 4)

        nc_w = lax.div(nchunk - 1 - w, NW) + 1

        def chunk(i, _):
            base = (w + NW * i) * CHQ
            pltpu.sync_copy(row_h.at[pl.ds(base, CHQ)], ib)
            pltpu.sync_copy(tr_h.at[pl.ds(base, CHQ)], tbuf)

            def quad(q, _):
                qidx = qoff + q * 4
                rv = plsc.load_gather(ib, [qidx])
                val = plsc.load_gather(tbuf, [qidx, cmod])
                plsc.addupdate_scatter(acc, [rv * 4 + cmod], val)
                return 0
            lax.fori_loop(0, CHQ // 4, quad, 0)
            return 0

        lax.fori_loop(0, nc_w, chunk, 0)
        pltpu.sync_copy(acc, dxp_h.at[pl.ds(w * n * 4, n * 4)])

    (dxp,) = k(row, trans4)
    return dxp.reshape(NW, n * 4)


def _dxr_body(dxp_ref, out_ref):
    acc = dxp_ref[0]
    for i in range(1, NW):
        acc = acc + dxp_ref[i]
    out_ref[...] = acc


def _dx_reduce(dxp_flat, n):
    total = dxp_flat.shape[1]
    return pl.pallas_call(
        _dxr_body,
        grid=(1,),
        in_specs=[pl.BlockSpec((NW, total), lambda i: (0, 0))],
        out_specs=pl.BlockSpec((total,), lambda i: (0,)),
        out_shape=jax.ShapeDtypeStruct((total,), jnp.float32),
    )(dxp_flat)


def _node_body(h_ref, coord_ref, aggp_ref, dx_ref, wn1_ref, bn1_ref,
               wn2_ref, bn2_ref, ho_ref, co_ref):
    hb = h_ref[...]
    agg = aggp_ref[0] + aggp_ref[1]
    z = _silu(jnp.dot(hb, wn1_ref[0:D, :], preferred_element_type=jnp.float32)
              + jnp.dot(agg, wn1_ref[D:2 * D, :], preferred_element_type=jnp.float32)
              + bn1_ref[...])
    ho_ref[...] = hb + jnp.dot(z, wn2_ref[...], preferred_element_type=jnp.float32) \
        + bn2_ref[...]
    co_ref[...] = coord_ref[...] + dx_ref[:, 0:3]


def _node_mlp(h, coord, aggp, dxp, wn1, bn1, wn2, bn2):
    n = h.shape[0]
    bn = 2000
    full = lambda shape: pl.BlockSpec(shape, lambda i: tuple(0 for _ in shape))
    return pl.pallas_call(
        _node_body,
        grid=(n // bn,),
        in_specs=[
            pl.BlockSpec((bn, D), lambda i: (i, 0)),
            pl.BlockSpec((bn, 3), lambda i: (i, 0)),
            pl.BlockSpec((NC, bn, H), lambda i: (0, i, 0)),
            pl.BlockSpec((bn, 4), lambda i: (i, 0)),
            full((2 * D, H)),
            full((1, H)),
            full((H, D)),
            full((1, D)),
        ],
        out_specs=[
            pl.BlockSpec((bn, D), lambda i: (i, 0)),
            pl.BlockSpec((bn, 3), lambda i: (i, 0)),
        ],
        out_shape=[
            jax.ShapeDtypeStruct((n, D), jnp.float32),
            jax.ShapeDtypeStruct((n, 3), jnp.float32),
        ],
    )(h, coord, aggp, dxp, wn1, bn1, wn2, bn2)


def kernel(h, edge_index, coord, edge_attr,
           We_w1, We_b1, We_w2, We_b2,
           Wn_w1, Wn_b1, Wn_w2, Wn_b2,
           Wc_w1, Wc_b1, Wc_w2):
    n = h.shape[0]
    e = edge_index.shape[1]
    row = edge_index[0]
    col = edge_index[1]

    wa = We_w1[0:D, :]
    wb = We_w1[D:2 * D, :]
    wr = We_w1[2 * D:2 * D + 1, :]
    we = We_w1[2 * D + 1:, :]

    A, B = _precompute(h, wa, wb)
    cx = coord[:, 0]
    cy = coord[:, 1]
    cz = coord[:, 2]

    P, cd4 = _sc_gather(A, B, edge_index, cx, cy, cz)
    rad = cd4[3]
    cds = jnp.concatenate(
        [cd4[0:3], jnp.zeros((1, e), jnp.float32)], axis=0).T

    m, trans4 = _edge_mlp(P, rad.reshape(e, 1), edge_attr, cds, we, wr,
                          We_b1.reshape(1, H), We_w2, We_b2.reshape(1, H),
                          Wc_w1, Wc_b1.reshape(1, H), Wc_w2)

    aggp = _sc_scatter(m, row, n)
    dxp_flat = _sc_dx(row, trans4, n)
    dx2 = _dx_reduce(dxp_flat, n).reshape(n, 4)

    h_out, coord_out = _node_mlp(h, coord, aggp, dx2, Wn_w1,
                                 Wn_b1.reshape(1, H), Wn_w2,
                                 Wn_b2.reshape(1, D))
    return h_out, coord_out, m


# speedup vs baseline: 4.8219x; 1.0413x over previous
import functools

import jax
import jax.numpy as jnp
from jax import lax
from jax.experimental import pallas as pl
from jax.experimental.pallas import tpu as pltpu
from jax.experimental.pallas import tpu_sc as plsc

D = 128
H = 128
DE = 16

NC = 2
NS = 16
L = 16
NW = NC * NS

CH = 256


def _silu(x):
    return 0.5 * x * (1.0 + jnp.tanh(0.5 * x))


def _pre_body(h_ref, wa_ref, wb_ref, a_ref, b_ref):
    hb = h_ref[...]
    a_ref[...] = jnp.dot(hb, wa_ref[...], preferred_element_type=jnp.float32)
    b_ref[...] = jnp.dot(hb, wb_ref[...], preferred_element_type=jnp.float32)


def _precompute(h, wa, wb):
    n = h.shape[0]
    bn = 2000
    return pl.pallas_call(
        _pre_body,
        grid=(n // bn,),
        in_specs=[
            pl.BlockSpec((bn, D), lambda i: (i, 0)),
            pl.BlockSpec((D, H), lambda i: (0, 0)),
            pl.BlockSpec((D, H), lambda i: (0, 0)),
        ],
        out_specs=[
            pl.BlockSpec((bn, H), lambda i: (i, 0)),
            pl.BlockSpec((bn, H), lambda i: (i, 0)),
        ],
        out_shape=[
            jax.ShapeDtypeStruct((n, H), jnp.float32),
            jax.ShapeDtypeStruct((n, H), jnp.float32),
        ],
    )(h, wa, wb)


CHG = 128


def _sc_gather(A, B, ei, cx, cy, cz):
    e = ei.shape[1]
    n = A.shape[0]
    nchunk = e // CHG
    nit = (nchunk + NW - 1) // NW
    npair = (nit + 1) // 2
    mesh = plsc.VectorSubcoreMesh(core_axis_name="c", subcore_axis_name="s")

    @functools.partial(
        pl.kernel,
        out_type=[
            jax.ShapeDtypeStruct((e, H), jnp.float32),
            jax.ShapeDtypeStruct((4, e), jnp.float32),
        ],
        mesh=mesh,
        scratch_types=[
            pltpu.VMEM((2, CHG), jnp.int32),
            pltpu.VMEM((2, CHG), jnp.int32),
            pltpu.VMEM((CHG, H), jnp.float32),
            pltpu.VMEM((CHG, H), jnp.float32),
            pltpu.VMEM((CHG, H), jnp.float32),
            pltpu.VMEM((CHG, H), jnp.float32),
            pltpu.VMEM((n,), jnp.float32),
            pltpu.VMEM((n,), jnp.float32),
            pltpu.VMEM((n,), jnp.float32),
            pltpu.VMEM((4, CHG), jnp.float32),
            pltpu.SemaphoreType.DMA,
            pltpu.SemaphoreType.DMA,
        ],
        compiler_params=pltpu.CompilerParams(needs_layout_passes=False),
    )
    def k(a_h, b_h, ei_h, cx_h, cy_h, cz_h,
          p_h, cd_h,
          ibuf0, ibuf1, abuf0, abuf1, bbuf0, bbuf1,
          cxt, cyt, czt, buf4, sem0, sem1):
        w = lax.axis_index("s") * NC + lax.axis_index("c")
        pltpu.sync_copy(cx_h, cxt)
        pltpu.sync_copy(cy_h, cyt)
        pltpu.sync_copy(cz_h, czt)

        sets = ((ibuf0, abuf0, bbuf0, sem0), (ibuf1, abuf1, bbuf1, sem1))

        def fire(c, si):
            ibuf, abuf, bbuf, sem = sets[si]
            base = c * CHG
            pltpu.sync_copy(ei_h.at[:, pl.ds(base, CHG)], ibuf)
            pltpu.async_copy(a_h.at[ibuf.at[0]], abuf, sem)
            pltpu.async_copy(b_h.at[ibuf.at[1]], bbuf, sem)

        def consume(c, si):
            ibuf, abuf, bbuf, sem = sets[si]
            base = c * CHG
            pltpu.make_async_copy(a_h.at[ibuf.at[0]], abuf, sem).wait()
            pltpu.make_async_copy(b_h.at[ibuf.at[1]], bbuf, sem).wait()

            def addrow(r, _):
                for j in range(H // L):
                    sl = pl.ds(j * L, L)
                    abuf[r, sl] = abuf[r, sl] + bbuf[r, sl]
                return 0
            lax.fori_loop(0, CHG, addrow, 0)

            for g in range(CHG // L):
                sl = pl.ds(g * L, L)
                ri = ibuf[0, sl]
                ci = ibuf[1, sl]
                xv = plsc.load_gather(cxt, [ri]) - plsc.load_gather(cxt, [ci])
                yv = plsc.load_gather(cyt, [ri]) - plsc.load_gather(cyt, [ci])
                zv = plsc.load_gather(czt, [ri]) - plsc.load_gather(czt, [ci])
                buf4[0, sl] = xv
                buf4[1, sl] = yv
                buf4[2, sl] = zv
                buf4[3, sl] = xv * xv + yv * yv + zv * zv

            pltpu.sync_copy(abuf, p_h.at[pl.ds(base, CHG)])
            pltpu.sync_copy(buf4, cd_h.at[:, pl.ds(base, CHG)])

        def cidx(i):
            return lax.rem(w + NW * i, nchunk)

        fire(cidx(0), 0)

        def body(p, _):
            fire(cidx(2 * p + 1), 1)
            consume(cidx(2 * p), 0)
            fire(cidx(2 * p + 2), 0)
            consume(cidx(2 * p + 1), 1)
            return 0

        lax.fori_loop(0, npair, body, 0)
        ibuf, abuf, bbuf, sem = sets[0]
        pltpu.make_async_copy(a_h.at[ibuf.at[0]], abuf, sem).wait()
        pltpu.make_async_copy(b_h.at[ibuf.at[1]], bbuf, sem).wait()

    return k(A, B, ei, cx, cy, cz)


def _edge_body(p_ref, rad_ref, ea_ref, cds_ref, we_ref, wr_ref, b1_ref,
               w2_ref, b2_ref, wc1_ref, bc1_ref, wc2_ref, m_ref, tr_ref):
    rad = rad_ref[...]
    pre = (p_ref[...].astype(jnp.float32)
           + jnp.dot(ea_ref[...], we_ref[...], preferred_element_type=jnp.float32)
           + rad * wr_ref[...] + b1_ref[...])
    m1 = _silu(pre).astype(jnp.bfloat16)
    m = _silu(jnp.dot(m1, w2_ref[...].astype(jnp.bfloat16),
                      preferred_element_type=jnp.float32) + b2_ref[...])
    mb = m.astype(jnp.bfloat16)
    ch = _silu(jnp.dot(mb, wc1_ref[...].astype(jnp.bfloat16),
                       preferred_element_type=jnp.float32)
               + bc1_ref[...]).astype(jnp.bfloat16)
    s = jnp.dot(ch, wc2_ref[...].astype(jnp.bfloat16),
                preferred_element_type=jnp.float32)
    m_ref[...] = m
    tr_ref[...] = cds_ref[...] * (s * lax.rsqrt(rad + 1e-8))


def _edge_mlp(P, rad2, ea, cds, we, wr, b1, w2, b2, wc1, bc1, wc2):
    e = P.shape[0]
    be = 2000
    full = lambda shape: pl.BlockSpec(shape, lambda i: tuple(0 for _ in shape))
    return pl.pallas_call(
        _edge_body,
        grid=(e // be,),
        in_specs=[
            pl.BlockSpec((be, H), lambda i: (i, 0)),
            pl.BlockSpec((be, 1), lambda i: (i, 0)),
            pl.BlockSpec((be, DE), lambda i: (i, 0)),
            pl.BlockSpec((be, 4), lambda i: (i, 0)),
            full((DE, H)),
            full((1, H)),
            full((1, H)),
            full((H, H)),
            full((1, H)),
            full((H, H)),
            full((1, H)),
            full((H, 1)),
        ],
        out_specs=[
            pl.BlockSpec((be, H), lambda i: (i, 0)),
            pl.BlockSpec((be, 4), lambda i: (i, 0)),
        ],
        out_shape=[
            jax.ShapeDtypeStruct((e, H), jnp.float32),
            jax.ShapeDtypeStruct((e, 4), jnp.float32),
        ],
    )(P, rad2, ea, cds, we, wr, b1, w2, b2, wc1, bc1, wc2)


def _cdiv(a, b):
    return (a + b - 1) // b


RB = 200
ZB = 80


DXW = 16
CHS = 128


def _sc_scatter(m, row, n):
    e = row.shape[0]
    nchunk = e // CHS
    nit = (nchunk + NW - 1) // NW
    npair = (nit + 1) // 2
    nzb = n // ZB
    mesh = plsc.VectorSubcoreMesh(core_axis_name="c", subcore_axis_name="s")

    @functools.partial(
        pl.kernel,
        out_type=[
            jax.ShapeDtypeStruct((NC, n, H), jnp.float32),
        ],
        mesh=mesh,
        scratch_types=[
            pltpu.VMEM((CHS, H), jnp.float32),
            pltpu.VMEM((CHS, H), jnp.float32),
            pltpu.VMEM((1, 128), jnp.int32),
            pltpu.VMEM((1, 128), jnp.int32),
            pltpu.VMEM_SHARED((n, H), jnp.float32),
            pltpu.SemaphoreType.DMA,
            pltpu.SemaphoreType.DMA,
            pltpu.SemaphoreType.DMA,
            pltpu.SemaphoreType.DMA,
        ],
        compiler_params=pltpu.CompilerParams(needs_layout_passes=False),
    )
    def k(m_h, row_h, aggp_h, mbuf0, mbuf1, ibuf0, ibuf1, agg_acc,
          sl0, sl1, sa0, sa1):
        cid = lax.axis_index("c")
        sid = lax.axis_index("s")
        w = sid * NC + cid

        zv = jnp.zeros((L,), jnp.float32)

        def zrow(r, _):
            for j in range(H // L):
                mbuf0[r, pl.ds(j * L, L)] = zv
            return 0
        lax.fori_loop(0, CHS, zrow, 0)

        nz_w = lax.div(nzb - 1 - sid, NS) + 1

        def zchunk(i, _):
            kk = sid + NS * i
            pltpu.sync_copy(mbuf0.at[pl.ds(0, ZB)],
                            agg_acc.at[pl.ds(kk * ZB, ZB)])
            return 0
        lax.fori_loop(0, nz_w, zchunk, 0)
        plsc.subcore_barrier()

        sets = ((mbuf0, ibuf0, sl0, sa0), (mbuf1, ibuf1, sl1, sa1))

        def cidx(i):
            return lax.rem(w + NW * i, nchunk)

        def fire(c, si):
            mbuf, ibuf, sl, sa = sets[si]
            base = c * CHS
            pltpu.sync_copy(row_h.at[pl.ds(base, CHS)], ibuf.at[0])
            pltpu.async_copy(m_h.at[pl.ds(base, CHS)], mbuf, sl)

        def consume(si):
            mbuf, ibuf, sl, sa = sets[si]
            pltpu.make_async_copy(m_h.at[pl.ds(0, CHS)], mbuf, sl).wait()
            pltpu.async_copy(mbuf, agg_acc.at[ibuf.at[0]], sa, add=True)

        def drain_add(si):
            mbuf, ibuf, sl, sa = sets[si]
            pltpu.make_async_copy(mbuf, agg_acc.at[ibuf.at[0]], sa).wait()

        npair = (nchunk // NW) // 2 - 1
        fire(cidx(0), 0)

        def body(p, _):
            fire(w + NW * (2 * p + 1), 1)
            consume(0)
            drain_add(0)
            fire(w + NW * (2 * p + 2), 0)
            consume(1)
            drain_add(1)
            return 0

        lax.fori_loop(0, npair, body, 0)
        consume(0)
        drain_add(0)

        nrem = lax.div(nchunk - 1 - w, NW) - 2 * npair

        def tail(i, _):
            fire(w + NW * (2 * npair + 1 + i), 0)
            consume(0)
            drain_add(0)
            return 0
        lax.fori_loop(0, nrem, tail, 0)
        plsc.subcore_barrier()

        def wchunk(i, _):
            kk = sid + NS * i
            pltpu.sync_copy(agg_acc.at[pl.ds(kk * ZB, ZB)],
                            mbuf0.at[pl.ds(0, ZB)])
            pltpu.sync_copy(mbuf0.at[pl.ds(0, ZB)],
                            aggp_h.at[cid, pl.ds(kk * ZB, ZB)])
            return 0
        lax.fori_loop(0, nz_w, wchunk, 0)

    (aggp,) = k(m, row)
    return aggp


CHQ = 512


def _sc_dx(row, trans4, n):
    e = row.shape[0]
    nchunk = e // CHQ
    mesh = plsc.VectorSubcoreMesh(core_axis_name="c", subcore_axis_name="s")

    @functools.partial(
        pl.kernel,
        out_type=[
            jax.ShapeDtypeStruct((NW * n * 4,), jnp.float32),
        ],
        mesh=mesh,
        scratch_types=[
            pltpu.VMEM((CHQ,), jnp.int32),
            pltpu.VMEM((CHQ, 4), jnp.float32),
            pltpu.VMEM((n * 4,), jnp.float32),
        ],
        compiler_params=pltpu.CompilerParams(needs_layout_passes=False),
    )
    def k(row_h, tr_h, dxp_h, ib, tbuf, acc):
        cid = lax.axis_index("c")
        sid = lax.axis_index("s")
        w = sid * NC + cid

        zv = jnp.zeros((L,), jnp.float32)

        def zflat(i, _):
            acc[pl.ds(i * L, L)] = zv
            return 0
        lax.fori_loop(0, n * 4 // L, zflat, 0)

        iota = lax.iota(jnp.int32, L)
        qoff = lax.div(iota, 4)
        cmod = lax.rem(iota, 4)

        nc_w = lax.div(nchunk - 1 - w, NW) + 1

        def chunk(i, _):
            base = (w + NW * i) * CHQ
            pltpu.sync_copy(row_h.at[pl.ds(base, CHQ)], ib)
            pltpu.sync_copy(tr_h.at[pl.ds(base, CHQ)], tbuf)

            def quad(q, _):
                qidx = qoff + q * 4
                rv = plsc.load_gather(ib, [qidx])
                val = plsc.load_gather(tbuf, [qidx, cmod])
                plsc.addupdate_scatter(acc, [rv * 4 + cmod], val)
                return 0
            lax.fori_loop(0, CHQ // 4, quad, 0)
            return 0

        lax.fori_loop(0, nc_w, chunk, 0)
        pltpu.sync_copy(acc, dxp_h.at[pl.ds(w * n * 4, n * 4)])

    (dxp,) = k(row, trans4)
    return dxp.reshape(NW, n * 4)


def _dxr_body(dxp_ref, out_ref):
    acc = dxp_ref[0]
    for i in range(1, NW):
        acc = acc + dxp_ref[i]
    out_ref[...] = acc


def _dx_reduce(dxp_flat, n):
    total = dxp_flat.shape[1]
    return pl.pallas_call(
        _dxr_body,
        grid=(1,),
        in_specs=[pl.BlockSpec((NW, total), lambda i: (0, 0))],
        out_specs=pl.BlockSpec((total,), lambda i: (0,)),
        out_shape=jax.ShapeDtypeStruct((total,), jnp.float32),
    )(dxp_flat)


def _node_body(h_ref, coord_ref, aggp_ref, dx_ref, wn1_ref, bn1_ref,
               wn2_ref, bn2_ref, ho_ref, co_ref):
    hb = h_ref[...]
    agg = aggp_ref[0] + aggp_ref[1]
    z = _silu(jnp.dot(hb, wn1_ref[0:D, :], preferred_element_type=jnp.float32)
              + jnp.dot(agg, wn1_ref[D:2 * D, :], preferred_element_type=jnp.float32)
              + bn1_ref[...])
    ho_ref[...] = hb + jnp.dot(z, wn2_ref[...], preferred_element_type=jnp.float32) \
        + bn2_ref[...]
    co_ref[...] = coord_ref[...] + dx_ref[:, 0:3]


def _node_mlp(h, coord, aggp, dxp, wn1, bn1, wn2, bn2):
    n = h.shape[0]
    bn = 2000
    full = lambda shape: pl.BlockSpec(shape, lambda i: tuple(0 for _ in shape))
    return pl.pallas_call(
        _node_body,
        grid=(n // bn,),
        in_specs=[
            pl.BlockSpec((bn, D), lambda i: (i, 0)),
            pl.BlockSpec((bn, 3), lambda i: (i, 0)),
            pl.BlockSpec((NC, bn, H), lambda i: (0, i, 0)),
            pl.BlockSpec((bn, 4), lambda i: (i, 0)),
            full((2 * D, H)),
            full((1, H)),
            full((H, D)),
            full((1, D)),
        ],
        out_specs=[
            pl.BlockSpec((bn, D), lambda i: (i, 0)),
            pl.BlockSpec((bn, 3), lambda i: (i, 0)),
        ],
        out_shape=[
            jax.ShapeDtypeStruct((n, D), jnp.float32),
            jax.ShapeDtypeStruct((n, 3), jnp.float32),
        ],
    )(h, coord, aggp, dxp, wn1, bn1, wn2, bn2)


def kernel(h, edge_index, coord, edge_attr,
           We_w1, We_b1, We_w2, We_b2,
           Wn_w1, Wn_b1, Wn_w2, Wn_b2,
           Wc_w1, Wc_b1, Wc_w2):
    n = h.shape[0]
    e = edge_index.shape[1]
    row = edge_index[0]
    col = edge_index[1]

    wa = We_w1[0:D, :]
    wb = We_w1[D:2 * D, :]
    wr = We_w1[2 * D:2 * D + 1, :]
    we = We_w1[2 * D + 1:, :]

    A, B = _precompute(h, wa, wb)
    cx = coord[:, 0]
    cy = coord[:, 1]
    cz = coord[:, 2]

    P, cd4 = _sc_gather(A, B, edge_index, cx, cy, cz)
    rad = cd4[3]
    cds = jnp.concatenate(
        [cd4[0:3], jnp.zeros((1, e), jnp.float32)], axis=0).T

    m, trans4 = _edge_mlp(P, rad.reshape(e, 1), edge_attr, cds, we, wr,
                          We_b1.reshape(1, H), We_w2, We_b2.reshape(1, H),
                          Wc_w1, Wc_b1.reshape(1, H), Wc_w2)

    aggp = _sc_scatter(m, row, n)
    dxp_flat = _sc_dx(row, trans4, n)
    dx2 = _dx_reduce(dxp_flat, n).reshape(n, 4)

    h_out, coord_out = _node_mlp(h, coord, aggp, dx2, Wn_w1,
                                 Wn_b1.reshape(1, H), Wn_w2,
                                 Wn_b2.reshape(1, D))
    return h_out, coord_out, m


# speedup vs baseline: 4.8252x; 1.0007x over previous
import functools

import jax
import jax.numpy as jnp
from jax import lax
from jax.experimental import pallas as pl
from jax.experimental.pallas import tpu as pltpu
from jax.experimental.pallas import tpu_sc as plsc

D = 128
H = 128
DE = 16

NC = 2
NS = 16
L = 16
NW = NC * NS


def _silu(x):
    return 0.5 * x * (1.0 + jnp.tanh(0.5 * x))


def _pre_body(h_ref, wa_ref, wb_ref, a_ref, b_ref):
    hb = h_ref[...]
    a_ref[...] = jnp.dot(hb, wa_ref[...], preferred_element_type=jnp.float32)
    b_ref[...] = jnp.dot(hb, wb_ref[...], preferred_element_type=jnp.float32)


def _precompute(h, wa, wb):
    n = h.shape[0]
    bn = 2000
    return pl.pallas_call(
        _pre_body,
        grid=(n // bn,),
        in_specs=[
            pl.BlockSpec((bn, D), lambda i: (i, 0)),
            pl.BlockSpec((D, H), lambda i: (0, 0)),
            pl.BlockSpec((D, H), lambda i: (0, 0)),
        ],
        out_specs=[
            pl.BlockSpec((bn, H), lambda i: (i, 0)),
            pl.BlockSpec((bn, H), lambda i: (i, 0)),
        ],
        out_shape=[
            jax.ShapeDtypeStruct((n, H), jnp.float32),
            jax.ShapeDtypeStruct((n, H), jnp.float32),
        ],
    )(h, wa, wb)


CHG = 128


def _sc_gather(A, B, ei, cx, cy, cz):
    e = ei.shape[1]
    n = A.shape[0]
    nchunk = e // CHG
    nit = (nchunk + NW - 1) // NW
    npair = (nit + 1) // 2
    mesh = plsc.VectorSubcoreMesh(core_axis_name="c", subcore_axis_name="s")

    @functools.partial(
        pl.kernel,
        out_type=[
            jax.ShapeDtypeStruct((e, H), jnp.float32),
            jax.ShapeDtypeStruct((4, e), jnp.float32),
        ],
        mesh=mesh,
        scratch_types=[
            pltpu.VMEM((2, CHG), jnp.int32),
            pltpu.VMEM((2, CHG), jnp.int32),
            pltpu.VMEM((CHG, H), jnp.float32),
            pltpu.VMEM((CHG, H), jnp.float32),
            pltpu.VMEM((CHG, H), jnp.float32),
            pltpu.VMEM((CHG, H), jnp.float32),
            pltpu.VMEM((n,), jnp.float32),
            pltpu.VMEM((n,), jnp.float32),
            pltpu.VMEM((n,), jnp.float32),
            pltpu.VMEM((4, CHG), jnp.float32),
            pltpu.SemaphoreType.DMA,
            pltpu.SemaphoreType.DMA,
        ],
        compiler_params=pltpu.CompilerParams(needs_layout_passes=False),
    )
    def k(a_h, b_h, ei_h, cx_h, cy_h, cz_h,
          p_h, cd_h,
          ibuf0, ibuf1, abuf0, abuf1, bbuf0, bbuf1,
          cxt, cyt, czt, buf4, sem0, sem1):
        w = lax.axis_index("s") * NC + lax.axis_index("c")
        pltpu.sync_copy(cx_h, cxt)
        pltpu.sync_copy(cy_h, cyt)
        pltpu.sync_copy(cz_h, czt)

        sets = ((ibuf0, abuf0, bbuf0, sem0), (ibuf1, abuf1, bbuf1, sem1))

        def fire(c, si):
            ibuf, abuf, bbuf, sem = sets[si]
            base = c * CHG
            pltpu.sync_copy(ei_h.at[:, pl.ds(base, CHG)], ibuf)
            pltpu.async_copy(a_h.at[ibuf.at[0]], abuf, sem)
            pltpu.async_copy(b_h.at[ibuf.at[1]], bbuf, sem)

        def consume(c, si):
            ibuf, abuf, bbuf, sem = sets[si]
            base = c * CHG
            pltpu.make_async_copy(a_h.at[ibuf.at[0]], abuf, sem).wait()
            pltpu.make_async_copy(b_h.at[ibuf.at[1]], bbuf, sem).wait()

            def addrow(r, _):
                for j in range(H // L):
                    sl = pl.ds(j * L, L)
                    abuf[r, sl] = abuf[r, sl] + bbuf[r, sl]
                return 0
            lax.fori_loop(0, CHG, addrow, 0)

            for g in range(CHG // L):
                sl = pl.ds(g * L, L)
                ri = ibuf[0, sl]
                ci = ibuf[1, sl]
                xv = plsc.load_gather(cxt, [ri]) - plsc.load_gather(cxt, [ci])
                yv = plsc.load_gather(cyt, [ri]) - plsc.load_gather(cyt, [ci])
                zv = plsc.load_gather(czt, [ri]) - plsc.load_gather(czt, [ci])
                buf4[0, sl] = xv
                buf4[1, sl] = yv
                buf4[2, sl] = zv
                buf4[3, sl] = xv * xv + yv * yv + zv * zv

            pltpu.sync_copy(abuf, p_h.at[pl.ds(base, CHG)])
            pltpu.sync_copy(buf4, cd_h.at[:, pl.ds(base, CHG)])

        def cidx(i):
            return lax.rem(w + NW * i, nchunk)

        fire(cidx(0), 0)

        def body(p, _):
            fire(cidx(2 * p + 1), 1)
            consume(cidx(2 * p), 0)
            fire(cidx(2 * p + 2), 0)
            consume(cidx(2 * p + 1), 1)
            return 0

        lax.fori_loop(0, npair, body, 0)
        ibuf, abuf, bbuf, sem = sets[0]
        pltpu.make_async_copy(a_h.at[ibuf.at[0]], abuf, sem).wait()
        pltpu.make_async_copy(b_h.at[ibuf.at[1]], bbuf, sem).wait()

    return k(A, B, ei, cx, cy, cz)


def _edge_body(p_ref, rad_ref, ea_ref, cds_ref, we_ref, wr_ref, b1_ref,
               w2_ref, b2_ref, wc1_ref, bc1_ref, wc2_ref, m_ref, tr_ref):
    rad = rad_ref[...]
    pre = (p_ref[...].astype(jnp.float32)
           + jnp.dot(ea_ref[...], we_ref[...], preferred_element_type=jnp.float32)
           + rad * wr_ref[...] + b1_ref[...])
    m1 = _silu(pre).astype(jnp.bfloat16)
    m = _silu(jnp.dot(m1, w2_ref[...].astype(jnp.bfloat16),
                      preferred_element_type=jnp.float32) + b2_ref[...])
    mb = m.astype(jnp.bfloat16)
    ch = _silu(jnp.dot(mb, wc1_ref[...].astype(jnp.bfloat16),
                       preferred_element_type=jnp.float32)
               + bc1_ref[...]).astype(jnp.bfloat16)
    s = jnp.dot(ch, wc2_ref[...].astype(jnp.bfloat16),
                preferred_element_type=jnp.float32)
    m_ref[...] = m
    tr_ref[...] = cds_ref[...] * (s * lax.rsqrt(rad + 1e-8))


def _edge_mlp(P, rad2, ea, cds, we, wr, b1, w2, b2, wc1, bc1, wc2):
    e = P.shape[0]
    be = 2000
    full = lambda shape: pl.BlockSpec(shape, lambda i: tuple(0 for _ in shape))
    return pl.pallas_call(
        _edge_body,
        grid=(e // be,),
        in_specs=[
            pl.BlockSpec((be, H), lambda i: (i, 0)),
            pl.BlockSpec((be, 1), lambda i: (i, 0)),
            pl.BlockSpec((be, DE), lambda i: (i, 0)),
            pl.BlockSpec((be, 4), lambda i: (i, 0)),
            full((DE, H)),
            full((1, H)),
            full((1, H)),
            full((H, H)),
            full((1, H)),
            full((H, H)),
            full((1, H)),
            full((H, 1)),
        ],
        out_specs=[
            pl.BlockSpec((be, H), lambda i: (i, 0)),
            pl.BlockSpec((be, 4), lambda i: (i, 0)),
        ],
        out_shape=[
            jax.ShapeDtypeStruct((e, H), jnp.float32),
            jax.ShapeDtypeStruct((e, 4), jnp.float32),
        ],
    )(P, rad2, ea, cds, we, wr, b1, w2, b2, wc1, bc1, wc2)


def _cdiv(a, b):
    return (a + b - 1) // b


ZB = 80


CHS = 128


def _sc_scatter(m, row, n):
    e = row.shape[0]
    nchunk = e // CHS
    nit = (nchunk + NW - 1) // NW
    npair = (nit + 1) // 2
    nzb = n // ZB
    mesh = plsc.VectorSubcoreMesh(core_axis_name="c", subcore_axis_name="s")

    @functools.partial(
        pl.kernel,
        out_type=[
            jax.ShapeDtypeStruct((NC, n, H), jnp.float32),
        ],
        mesh=mesh,
        scratch_types=[
            pltpu.VMEM((CHS, H), jnp.float32),
            pltpu.VMEM((CHS, H), jnp.float32),
            pltpu.VMEM((1, 128), jnp.int32),
            pltpu.VMEM((1, 128), jnp.int32),
            pltpu.VMEM_SHARED((n, H), jnp.float32),
            pltpu.SemaphoreType.DMA,
            pltpu.SemaphoreType.DMA,
            pltpu.SemaphoreType.DMA,
            pltpu.SemaphoreType.DMA,
        ],
        compiler_params=pltpu.CompilerParams(needs_layout_passes=False),
    )
    def k(m_h, row_h, aggp_h, mbuf0, mbuf1, ibuf0, ibuf1, agg_acc,
          sl0, sl1, sa0, sa1):
        cid = lax.axis_index("c")
        sid = lax.axis_index("s")
        w = sid * NC + cid

        zv = jnp.zeros((L,), jnp.float32)

        def zrow(r, _):
            for j in range(H // L):
                mbuf0[r, pl.ds(j * L, L)] = zv
            return 0
        lax.fori_loop(0, CHS, zrow, 0)

        nz_w = lax.div(nzb - 1 - sid, NS) + 1

        def zchunk(i, _):
            kk = sid + NS * i
            pltpu.sync_copy(mbuf0.at[pl.ds(0, ZB)],
                            agg_acc.at[pl.ds(kk * ZB, ZB)])
            return 0
        lax.fori_loop(0, nz_w, zchunk, 0)
        plsc.subcore_barrier()

        sets = ((mbuf0, ibuf0, sl0, sa0), (mbuf1, ibuf1, sl1, sa1))

        def cidx(i):
            return lax.rem(w + NW * i, nchunk)

        def fire(c, si):
            mbuf, ibuf, sl, sa = sets[si]
            base = c * CHS
            pltpu.sync_copy(row_h.at[pl.ds(base, CHS)], ibuf.at[0])
            pltpu.async_copy(m_h.at[pl.ds(base, CHS)], mbuf, sl)

        def consume(si):
            mbuf, ibuf, sl, sa = sets[si]
            pltpu.make_async_copy(m_h.at[pl.ds(0, CHS)], mbuf, sl).wait()
            pltpu.async_copy(mbuf, agg_acc.at[ibuf.at[0]], sa, add=True)

        def drain_add(si):
            mbuf, ibuf, sl, sa = sets[si]
            pltpu.make_async_copy(mbuf, agg_acc.at[ibuf.at[0]], sa).wait()

        npair = (nchunk // NW) // 2 - 1
        fire(cidx(0), 0)

        def body(p, _):
            fire(w + NW * (2 * p + 1), 1)
            consume(0)
            drain_add(0)
            fire(w + NW * (2 * p + 2), 0)
            consume(1)
            drain_add(1)
            return 0

        lax.fori_loop(0, npair, body, 0)
        consume(0)
        drain_add(0)

        nrem = lax.div(nchunk - 1 - w, NW) - 2 * npair

        def tail(i, _):
            fire(w + NW * (2 * npair + 1 + i), 0)
            consume(0)
            drain_add(0)
            return 0
        lax.fori_loop(0, nrem, tail, 0)
        plsc.subcore_barrier()

        def wchunk(i, _):
            kk = sid + NS * i
            pltpu.sync_copy(agg_acc.at[pl.ds(kk * ZB, ZB)],
                            mbuf0.at[pl.ds(0, ZB)])
            pltpu.sync_copy(mbuf0.at[pl.ds(0, ZB)],
                            aggp_h.at[cid, pl.ds(kk * ZB, ZB)])
            return 0
        lax.fori_loop(0, nz_w, wchunk, 0)

    (aggp,) = k(m, row)
    return aggp


CHQ = 512


def _sc_dx(row, trans4, n):
    e = row.shape[0]
    nchunk = e // CHQ
    mesh = plsc.VectorSubcoreMesh(core_axis_name="c", subcore_axis_name="s")

    @functools.partial(
        pl.kernel,
        out_type=[
            jax.ShapeDtypeStruct((NW * n * 4,), jnp.float32),
        ],
        mesh=mesh,
        scratch_types=[
            pltpu.VMEM((CHQ,), jnp.int32),
            pltpu.VMEM((CHQ, 4), jnp.float32),
            pltpu.VMEM((n * 4,), jnp.float32),
        ],
        compiler_params=pltpu.CompilerParams(needs_layout_passes=False),
    )
    def k(row_h, tr_h, dxp_h, ib, tbuf, acc):
        cid = lax.axis_index("c")
        sid = lax.axis_index("s")
        w = sid * NC + cid

        zv = jnp.zeros((L,), jnp.float32)

        def zflat(i, _):
            acc[pl.ds(i * L, L)] = zv
            return 0
        lax.fori_loop(0, n * 4 // L, zflat, 0)

        iota = lax.iota(jnp.int32, L)
        qoff = lax.div(iota, 4)
        cmod = lax.rem(iota, 4)

        nc_w = lax.div(nchunk - 1 - w, NW) + 1

        def chunk(i, _):
            base = (w + NW * i) * CHQ
            pltpu.sync_copy(row_h.at[pl.ds(base, CHQ)], ib)
            pltpu.sync_copy(tr_h.at[pl.ds(base, CHQ)], tbuf)

            def quad(q, _):
                qidx = qoff + q * 4
                rv = plsc.load_gather(ib, [qidx])
                val = plsc.load_gather(tbuf, [qidx, cmod])
                plsc.addupdate_scatter(acc, [rv * 4 + cmod], val)
                return 0
            lax.fori_loop(0, CHQ // 4, quad, 0)
            return 0

        lax.fori_loop(0, nc_w, chunk, 0)
        pltpu.sync_copy(acc, dxp_h.at[pl.ds(w * n * 4, n * 4)])

    (dxp,) = k(row, trans4)
    return dxp.reshape(NW, n * 4)


def _dxr_body(dxp_ref, out_ref):
    acc = dxp_ref[0]
    for i in range(1, NW):
        acc = acc + dxp_ref[i]
    out_ref[...] = acc


def _dx_reduce(dxp_flat, n):
    total = dxp_flat.shape[1]
    return pl.pallas_call(
        _dxr_body,
        grid=(1,),
        in_specs=[pl.BlockSpec((NW, total), lambda i: (0, 0))],
        out_specs=pl.BlockSpec((total,), lambda i: (0,)),
        out_shape=jax.ShapeDtypeStruct((total,), jnp.float32),
    )(dxp_flat)


def _node_body(h_ref, coord_ref, aggp_ref, dx_ref, wn1_ref, bn1_ref,
               wn2_ref, bn2_ref, ho_ref, co_ref):
    hb = h_ref[...]
    agg = aggp_ref[0] + aggp_ref[1]
    z = _silu(jnp.dot(hb, wn1_ref[0:D, :], preferred_element_type=jnp.float32)
              + jnp.dot(agg, wn1_ref[D:2 * D, :], preferred_element_type=jnp.float32)
              + bn1_ref[...])
    ho_ref[...] = hb + jnp.dot(z, wn2_ref[...], preferred_element_type=jnp.float32) \
        + bn2_ref[...]
    co_ref[...] = coord_ref[...] + dx_ref[:, 0:3]


def _node_mlp(h, coord, aggp, dxp, wn1, bn1, wn2, bn2):
    n = h.shape[0]
    bn = 2000
    full = lambda shape: pl.BlockSpec(shape, lambda i: tuple(0 for _ in shape))
    return pl.pallas_call(
        _node_body,
        grid=(n // bn,),
        in_specs=[
            pl.BlockSpec((bn, D), lambda i: (i, 0)),
            pl.BlockSpec((bn, 3), lambda i: (i, 0)),
            pl.BlockSpec((NC, bn, H), lambda i: (0, i, 0)),
            pl.BlockSpec((bn, 4), lambda i: (i, 0)),
            full((2 * D, H)),
            full((1, H)),
            full((H, D)),
            full((1, D)),
        ],
        out_specs=[
            pl.BlockSpec((bn, D), lambda i: (i, 0)),
            pl.BlockSpec((bn, 3), lambda i: (i, 0)),
        ],
        out_shape=[
            jax.ShapeDtypeStruct((n, D), jnp.float32),
            jax.ShapeDtypeStruct((n, 3), jnp.float32),
        ],
    )(h, coord, aggp, dxp, wn1, bn1, wn2, bn2)


def kernel(h, edge_index, coord, edge_attr,
           We_w1, We_b1, We_w2, We_b2,
           Wn_w1, Wn_b1, Wn_w2, Wn_b2,
           Wc_w1, Wc_b1, Wc_w2):
    n = h.shape[0]
    e = edge_index.shape[1]
    row = edge_index[0]
    col = edge_index[1]

    wa = We_w1[0:D, :]
    wb = We_w1[D:2 * D, :]
    wr = We_w1[2 * D:2 * D + 1, :]
    we = We_w1[2 * D + 1:, :]

    A, B = _precompute(h, wa, wb)
    cx = coord[:, 0]
    cy = coord[:, 1]
    cz = coord[:, 2]

    P, cd4 = _sc_gather(A, B, edge_index, cx, cy, cz)
    rad = cd4[3]
    cds = jnp.concatenate(
        [cd4[0:3], jnp.zeros((1, e), jnp.float32)], axis=0).T

    m, trans4 = _edge_mlp(P, rad.reshape(e, 1), edge_attr, cds, we, wr,
                          We_b1.reshape(1, H), We_w2, We_b2.reshape(1, H),
                          Wc_w1, Wc_b1.reshape(1, H), Wc_w2)

    aggp = _sc_scatter(m, row, n)
    dxp_flat = _sc_dx(row, trans4, n)
    dx2 = _dx_reduce(dxp_flat, n).reshape(n, 4)

    h_out, coord_out = _node_mlp(h, coord, aggp, dx2, Wn_w1,
                                 Wn_b1.reshape(1, H), Wn_w2,
                                 Wn_b2.reshape(1, D))
    return h_out, coord_out, m
